# baseline (speedup 1.0000x reference)
import dataclasses
import functools

import jax
import jax.numpy as jnp
import numpy as np
from jax import lax
from jax.experimental import pallas as pl
from jax.experimental.pallas import tpu as pltpu
from jax.experimental.pallas import tpu_sc as plsc

N = 10000
E = 160000
IN_DIM = 256
OUT_DIM = 64
HEADS = 8
C = OUT_DIM * HEADS

CH = 128
E_PAD = 163840
NCH = E_PAD // CH
NC, NS = 2, 16
NP = 10240
ROWS_PER_TILE = NP // NS

_MESH = plsc.VectorSubcoreMesh(core_axis_name="c", subcore_axis_name="s")

_SC_CP = pltpu.CompilerParams()
if "needs_layout_passes" in pltpu.CompilerParams.__dataclass_fields__:
    _SC_CP = dataclasses.replace(_SC_CP, needs_layout_passes=False)

_NBLK = 400
_EBLK = 2048


def _proj_body(q_ref, k_ref, v_ref, wq_ref, wk_ref, wv_ref, wr_ref,
               Q_ref, K_ref, V_ref, R_ref):
    Q_ref[...] = jnp.dot(q_ref[...], wq_ref[...],
                         preferred_element_type=jnp.float32)
    K_ref[...] = jnp.dot(k_ref[...], wk_ref[...],
                         preferred_element_type=jnp.float32)
    V_ref[...] = jnp.dot(v_ref[...], wv_ref[...],
                         preferred_element_type=jnp.float32)
    R_ref[...] = jnp.dot(q_ref[...], wr_ref[...],
                         preferred_element_type=jnp.float32)


def _proj(q, k, v, Wq, Wk, Wv, Wres):
    nb = N // _NBLK
    blk = lambda i: (i, 0)
    w_spec = pl.BlockSpec((IN_DIM, C), lambda i: (0, 0))
    return pl.pallas_call(
        _proj_body,
        grid=(nb,),
        in_specs=[pl.BlockSpec((_NBLK, IN_DIM), blk)] * 3 + [w_spec] * 4,
        out_specs=[pl.BlockSpec((_NBLK, C), blk)] * 4,
        out_shape=[jax.ShapeDtypeStruct((N, C), jnp.float32)] * 4,
    )(q, k, v, Wq, Wk, Wv, Wres)


def _scores_body(qg_ref, kg_ref, ef_ref, wep_ref, bep_ref, m_ref,
                 ex16_ref, exX_ref):
    prod = qg_ref[...] * kg_ref[...]
    s = jnp.dot(prod, m_ref[...], preferred_element_type=jnp.float32)
    s = s * (1.0 / float(OUT_DIM) ** 0.5)
    s = s + jnp.dot(ef_ref[...], wep_ref[...],
                    preferred_element_type=jnp.float32) + bep_ref[...]
    row = (pl.program_id(0) * _EBLK
           + lax.broadcasted_iota(jnp.int32, (_EBLK, 1), 0))
    s = jnp.where(row < E, s, -1e30)
    ex = jnp.exp(s)
    ex16_ref[...] = ex
    exX_ref[...] = jnp.concatenate(
        [jnp.broadcast_to(ex[:, h:h + 1], (_EBLK, OUT_DIM))
         for h in range(HEADS)], axis=1)


def _scores(Qg, Kg, ef, WeP, beP, M):
    nb = E_PAD // _EBLK
    blk = lambda i: (i, 0)
    return pl.pallas_call(
        _scores_body,
        grid=(nb,),
        in_specs=[pl.BlockSpec((_EBLK, C), blk),
                  pl.BlockSpec((_EBLK, C), blk),
                  pl.BlockSpec((_EBLK, 16), blk),
                  pl.BlockSpec((16, 16), lambda i: (0, 0)),
                  pl.BlockSpec((1, 16), lambda i: (0, 0)),
                  pl.BlockSpec((C, 16), lambda i: (0, 0))],
        out_specs=[pl.BlockSpec((_EBLK, 16), blk),
                   pl.BlockSpec((_EBLK, C), blk)],
        out_shape=[jax.ShapeDtypeStruct((E_PAD, 16), jnp.float32),
                   jax.ShapeDtypeStruct((E_PAD, C), jnp.float32)],
    )(Qg, Kg, ef, WeP, beP, M)


def _rden_body(dp_ref, out_ref):
    den = dp_ref[0] + dp_ref[1]
    out_ref[...] = 1.0 / jnp.maximum(den, 1e-30)


def _rden(dpart):
    return pl.pallas_call(
        _rden_body,
        out_shape=jax.ShapeDtypeStruct((NP, 128), jnp.float32),
    )(dpart)


def _head1_body(rst_ref, rd_ref, wo_ref, qres_ref, z_ref, s_ref, q_ref):
    rd = rd_ref[...]
    rscale = jnp.concatenate(
        [jnp.broadcast_to(rd[:, h:h + 1], (_NBLK, OUT_DIM))
         for h in range(HEADS)], axis=1)
    z = jnp.dot(rst_ref[...] * rscale, wo_ref[...],
                preferred_element_type=jnp.float32) + qres_ref[...]
    z_ref[...] = z

    @pl.when(pl.program_id(0) == 0)
    def _():
        s_ref[...] = jnp.zeros_like(s_ref)
        q_ref[...] = jnp.zeros_like(q_ref)

    s_ref[...] += jnp.sum(z, axis=0, keepdims=True)
    q_ref[...] += jnp.sum(z * z, axis=0, keepdims=True)


def _head1(rst, rdenN, Wo, Qres):
    nb = N // _NBLK
    blk = lambda i: (i, 0)
    acc = pl.BlockSpec((1, C), lambda i: (0, 0))
    return pl.pallas_call(
        _head1_body,
        grid=(nb,),
        in_specs=[pl.BlockSpec((_NBLK, C), blk),
                  pl.BlockSpec((_NBLK, 128), blk),
                  pl.BlockSpec((C, C), lambda i: (0, 0)),
                  pl.BlockSpec((_NBLK, C), blk)],
        out_specs=[pl.BlockSpec((_NBLK, C), blk), acc, acc],
        out_shape=[jax.ShapeDtypeStruct((N, C), jnp.float32),
                   jax.ShapeDtypeStruct((1, C), jnp.float32),
                   jax.ShapeDtypeStruct((1, C), jnp.float32)],
    )(rst, rdenN, Wo, Qres)


def _head2_body(z_ref, a1_ref, b1_ref, w1_ref, bf1_ref, w2_ref, bf2_ref,
                y_ref, s_ref, q_ref):
    zn = z_ref[...] * a1_ref[...] + b1_ref[...]
    h = jnp.maximum(jnp.dot(zn, w1_ref[...],
                            preferred_element_type=jnp.float32)
                    + bf1_ref[...], 0.0)
    y = jnp.dot(h, w2_ref[...],
                preferred_element_type=jnp.float32) + bf2_ref[...] + zn
    y_ref[...] = y

    @pl.when(pl.program_id(0) == 0)
    def _():
        s_ref[...] = jnp.zeros_like(s_ref)
        q_ref[...] = jnp.zeros_like(q_ref)

    s_ref[...] += jnp.sum(y, axis=0, keepdims=True)
    q_ref[...] += jnp.sum(y * y, axis=0, keepdims=True)


def _head2(Z, a1, b1, W1, bf1, W2, bf2):
    nb = N // _NBLK
    blk = lambda i: (i, 0)
    acc = pl.BlockSpec((1, C), lambda i: (0, 0))
    one = lambda shape: pl.BlockSpec(shape, lambda i: (0, 0))
    return pl.pallas_call(
        _head2_body,
        grid=(nb,),
        in_specs=[pl.BlockSpec((_NBLK, C), blk),
                  one((1, C)), one((1, C)),
                  one((C, 2 * C)), one((1, 2 * C)),
                  one((2 * C, C)), one((1, C))],
        out_specs=[pl.BlockSpec((_NBLK, C), blk), acc, acc],
        out_shape=[jax.ShapeDtypeStruct((N, C), jnp.float32),
                   jax.ShapeDtypeStruct((1, C), jnp.float32),
                   jax.ShapeDtypeStruct((1, C), jnp.float32)],
    )(Z, a1, b1, W1, bf1, W2, bf2)


def _bn2_body(y_ref, a2_ref, b2_ref, out_ref):
    out_ref[...] = y_ref[...] * a2_ref[...] + b2_ref[...]


def _bn2(Y, a2, b2):
    nb = N // _NBLK
    blk = lambda i: (i, 0)
    return pl.pallas_call(
        _bn2_body,
        grid=(nb,),
        in_specs=[pl.BlockSpec((_NBLK, C), blk),
                  pl.BlockSpec((1, C), lambda i: (0, 0)),
                  pl.BlockSpec((1, C), lambda i: (0, 0))],
        out_specs=pl.BlockSpec((_NBLK, C), blk),
        out_shape=jax.ShapeDtypeStruct((N, C), jnp.float32),
    )(Y, a2, b2)


def _sc_gather_body(Q_hbm, K_hbm, dstR_hbm, srcR_hbm, Qg_hbm, Kg_hbm,
                    ibuf, buf):
    c = lax.axis_index("c")
    s = lax.axis_index("s")
    w = s * NC + c
    lo = w * (NCH // (NC * NS))

    @pl.loop(0, NCH // (NC * NS), unroll=1)
    def _(i):
        r = lo + i
        base = r * CH
        pltpu.sync_copy(dstR_hbm.at[r], ibuf)
        pltpu.sync_copy(Q_hbm.at[ibuf.at[0]], buf)
        pltpu.sync_copy(buf, Qg_hbm.at[pl.ds(base, CH)])
        pltpu.sync_copy(srcR_hbm.at[r], ibuf)
        pltpu.sync_copy(K_hbm.at[ibuf.at[0]], buf)
        pltpu.sync_copy(buf, Kg_hbm.at[pl.ds(base, CH)])


def _sc_gather(Q, K, dstR, srcR):
    f = pl.kernel(
        _sc_gather_body,
        out_type=(jax.ShapeDtypeStruct((E_PAD, C), jnp.float32),
                  jax.ShapeDtypeStruct((E_PAD, C), jnp.float32)),
        mesh=_MESH,
        scratch_types=[pltpu.VMEM((1, CH), jnp.int32),
                       pltpu.VMEM((CH, C), jnp.float32)],
    )
    return f(Q, K, dstR, srcR)


def _sc_denom_body(ex16_hbm, dstR_hbm, z128_hbm, dpart_hbm,
                   ibuf, exbuf, padbuf, acc):
    c = lax.axis_index("c")
    s = lax.axis_index("s")
    row0 = s * ROWS_PER_TILE
    pltpu.sync_copy(z128_hbm, acc.at[pl.ds(row0, ROWS_PER_TILE)])
    pltpu.sync_copy(z128_hbm.at[pl.ds(0, CH)], padbuf)
    plsc.subcore_barrier()

    lo = c * (NCH // NC) + s * (NCH // (NC * NS))

    @pl.loop(0, NCH // (NC * NS), unroll=1)
    def _(i):
        r = lo + i
        pltpu.sync_copy(dstR_hbm.at[r], ibuf)
        pltpu.sync_copy(ex16_hbm.at[pl.ds(r * CH, CH)], exbuf)

        @pl.loop(0, CH)
        def _(j):
            padbuf[j, pl.ds(0, 16)] = exbuf[j, :]

        pltpu.sync_copy(padbuf, acc.at[ibuf.at[0]], add=True)

    plsc.subcore_barrier()
    pltpu.sync_copy(acc.at[pl.ds(row0, ROWS_PER_TILE)],
                    dpart_hbm.at[c, pl.ds(row0, ROWS_PER_TILE)])


def _sc_denom(ex16, dstR, z128):
    f = pl.kernel(
        _sc_denom_body,
        out_type=jax.ShapeDtypeStruct((NC, NP, 128), jnp.float32),
        mesh=_MESH,
        scratch_types=[pltpu.VMEM((1, CH), jnp.int32),
                       pltpu.VMEM((CH, 16), jnp.float32),
                       pltpu.VMEM((CH, 128), jnp.float32),
                       pltpu.VMEM_SHARED((NP, 128), jnp.float32)],
    )
    return f(ex16, dstR, z128)


def _sc_msg_body(VR_hbm, exX_hbm, dstR_hbm, srcR_hbm, z128_hbm,
                 rst_hbm, dbuf, sbuf, i2buf, axbuf, vbuf, acc):
    c = lax.axis_index("c")
    s = lax.axis_index("s")
    row0 = s * ROWS_PER_TILE
    lo = s * (NCH // NS)

    for p in range(2):
        g = c * 2 + p
        pltpu.sync_copy(z128_hbm, acc.at[pl.ds(row0, ROWS_PER_TILE)])
        plsc.subcore_barrier()

        @pl.loop(0, NCH // NS, unroll=1)
        def _(i):
            r = lo + i
            base = r * CH
            pltpu.sync_copy(dstR_hbm.at[r], dbuf)
            pltpu.sync_copy(srcR_hbm.at[r], sbuf)
            pltpu.sync_copy(exX_hbm.at[pl.ds(base, CH),
                                       pl.ds(g * 128, 128)], axbuf)
            for j in range(CH // 16):
                sv = sbuf[0, pl.ds(j * 16, 16)]
                i2buf[0, pl.ds(j * 16, 16)] = sv * 4 + g
            pltpu.sync_copy(VR_hbm.at[i2buf.at[0]], vbuf)

            @pl.loop(0, CH)
            def _(j):
                for t in range(8):
                    sl = pl.ds(t * 16, 16)
                    vbuf[j, sl] = vbuf[j, sl] * axbuf[j, sl]

            pltpu.sync_copy(vbuf, acc.at[dbuf.at[0]], add=True)

        plsc.subcore_barrier()
        pltpu.sync_copy(acc.at[pl.ds(row0, ROWS_PER_TILE)],
                        rst_hbm.at[pl.ds(row0, ROWS_PER_TILE),
                                   pl.ds(g * 128, 128)])
        plsc.subcore_barrier()


def _sc_msg(VR, exX, dstR, srcR, z128):
    f = pl.kernel(
        _sc_msg_body,
        out_type=jax.ShapeDtypeStruct((NP, C), jnp.float32),
        mesh=_MESH,
        scratch_types=[pltpu.VMEM((1, CH), jnp.int32),
                       pltpu.VMEM((1, CH), jnp.int32),
                       pltpu.VMEM((1, CH), jnp.int32),
                       pltpu.VMEM((CH, 128), jnp.float32),
                       pltpu.VMEM((CH, 128), jnp.float32),
                       pltpu.VMEM_SHARED((NP, 128), jnp.float32)],
    )
    return f(VR, exX, dstR, srcR, z128)


def kernel(q, k, v, edge_feat, edge_index, Wq, Wk, Wv, We, be, Wo, Wres,
           W1, bf1, W2, bf2, gamma1, beta1, gamma2, beta2):
    src = edge_index[0].astype(jnp.int32)
    dst = edge_index[1].astype(jnp.int32)
    pad = jnp.zeros((E_PAD - E,), jnp.int32)
    dstR = jnp.concatenate([dst, pad]).reshape(NCH, 1, CH)
    srcR = jnp.concatenate([src, pad]).reshape(NCH, 1, CH)
    efP = jnp.concatenate(
        [edge_feat, jnp.zeros((E_PAD - E, 16), jnp.float32)], axis=0)

    WeP = We
    WeP = jnp.concatenate([We, jnp.zeros((16, 8), jnp.float32)], axis=1)
    beP = jnp.concatenate([be, jnp.full((8,), -1e30, jnp.float32)])
    beP = beP.reshape(1, 16)

    m_np = np.zeros((C, 16), np.float32)
    for h in range(HEADS):
        m_np[h * OUT_DIM:(h + 1) * OUT_DIM, h] = 1.0
    M = jnp.asarray(m_np)

    z128 = jnp.zeros((ROWS_PER_TILE, 128), jnp.float32)

    Q, K, V, Qres = _proj(q, k, v, Wq, Wk, Wv, Wres)
    VR = V.reshape(N * 4, 128)

    Qg, Kg = _sc_gather(Q, K, dstR, srcR)
    ex16, exX = _scores(Qg, Kg, efP, WeP, beP, M)

    dpart = _sc_denom(ex16, dstR, z128)
    rden = _rden(dpart)

    exX, rden = lax.optimization_barrier((exX, rden))

    rst = _sc_msg(VR, exX, dstR, srcR, z128)
    rst = rst[:N]
    rdenN = rden[:N]

    Z, s1, q1 = _head1(rst, rdenN, Wo, Qres)
    mean1 = s1 / N
    var1 = q1 / N - mean1 * mean1
    a1 = gamma1 / jnp.sqrt(var1 + 1e-5)
    b1 = beta1 - mean1 * a1

    Y, s2, q2 = _head2(Z, a1, b1, W1, bf1.reshape(1, -1), W2,
                       bf2.reshape(1, -1))
    mean2 = s2 / N
    var2 = q2 / N - mean2 * mean2
    a2 = gamma2 / jnp.sqrt(var2 + 1e-5)
    b2 = beta2 - mean2 * a2

    return _bn2(Y, a2, b2)


# speedup vs baseline: 1.4323x; 1.4323x over previous
import dataclasses
import functools

import jax
import jax.numpy as jnp
import numpy as np
from jax import lax
from jax.experimental import pallas as pl
from jax.experimental.pallas import tpu as pltpu
from jax.experimental.pallas import tpu_sc as plsc

N = 10000
E = 160000
IN_DIM = 256
OUT_DIM = 64
HEADS = 8
C = OUT_DIM * HEADS

CH = 128
E_PAD = 160000
NCH = E_PAD // CH
NC, NS = 2, 16
NP = 10240
ROWS_PER_TILE = NP // NS

_MESH = plsc.VectorSubcoreMesh(core_axis_name="c", subcore_axis_name="s")

_SC_CP = pltpu.CompilerParams()
if "needs_layout_passes" in pltpu.CompilerParams.__dataclass_fields__:
    _SC_CP = dataclasses.replace(_SC_CP, needs_layout_passes=False)

_NBLK = 400
_EBLK = 2000


def _proj_body(q_ref, k_ref, v_ref, wq_ref, wk_ref, wv_ref, wr_ref,
               Q_ref, K_ref, V_ref, R_ref):
    Q_ref[...] = jnp.dot(q_ref[...], wq_ref[...],
                         preferred_element_type=jnp.float32)
    K_ref[...] = jnp.dot(k_ref[...], wk_ref[...],
                         preferred_element_type=jnp.float32)
    V_ref[...] = jnp.dot(v_ref[...], wv_ref[...],
                         preferred_element_type=jnp.float32)
    R_ref[...] = jnp.dot(q_ref[...], wr_ref[...],
                         preferred_element_type=jnp.float32)


def _proj(q, k, v, Wq, Wk, Wv, Wres):
    nb = N // _NBLK
    blk = lambda i: (i, 0)
    w_spec = pl.BlockSpec((IN_DIM, C), lambda i: (0, 0))
    return pl.pallas_call(
        _proj_body,
        grid=(nb,),
        in_specs=[pl.BlockSpec((_NBLK, IN_DIM), blk)] * 3 + [w_spec] * 4,
        out_specs=[pl.BlockSpec((_NBLK, C), blk)] * 4,
        out_shape=[jax.ShapeDtypeStruct((N, C), jnp.float32)] * 4,
    )(q, k, v, Wq, Wk, Wv, Wres)


def _scores_body(qg_ref, kg_ref, ef_ref, wep_ref, bep_ref, m_ref,
                 ex16_ref, exX_ref):
    prod = qg_ref[...] * kg_ref[...]
    s = jnp.dot(prod, m_ref[...], preferred_element_type=jnp.float32)
    s = s * (1.0 / float(OUT_DIM) ** 0.5)
    s = s + jnp.dot(ef_ref[...], wep_ref[...],
                    preferred_element_type=jnp.float32) + bep_ref[...]
    row = (pl.program_id(0) * _EBLK
           + lax.broadcasted_iota(jnp.int32, (_EBLK, 1), 0))
    s = jnp.where(row < E, s, -1e30)
    ex = jnp.exp(s)
    ex16_ref[...] = ex
    exX_ref[...] = jnp.concatenate(
        [jnp.broadcast_to(ex[:, h:h + 1], (_EBLK, OUT_DIM))
         for h in range(HEADS)], axis=1)


def _scores(Qg, Kg, ef, WeP, beP, M):
    nb = E_PAD // _EBLK
    blk = lambda i: (i, 0)
    return pl.pallas_call(
        _scores_body,
        grid=(nb,),
        in_specs=[pl.BlockSpec((_EBLK, C), blk),
                  pl.BlockSpec((_EBLK, C), blk),
                  pl.BlockSpec((_EBLK, 16), blk),
                  pl.BlockSpec((16, 16), lambda i: (0, 0)),
                  pl.BlockSpec((1, 16), lambda i: (0, 0)),
                  pl.BlockSpec((C, 16), lambda i: (0, 0))],
        out_specs=[pl.BlockSpec((_EBLK, 16), blk),
                   pl.BlockSpec((_EBLK, C), blk)],
        out_shape=[jax.ShapeDtypeStruct((E_PAD, 16), jnp.float32),
                   jax.ShapeDtypeStruct((E_PAD, C), jnp.float32)],
    )(Qg, Kg, ef, WeP, beP, M)


def _rden_body(dp_ref, out_ref):
    den = dp_ref[0] + dp_ref[1]
    out_ref[...] = 1.0 / jnp.maximum(den, 1e-30)


def _rden(dpart):
    return pl.pallas_call(
        _rden_body,
        out_shape=jax.ShapeDtypeStruct((NP, 128), jnp.float32),
    )(dpart)


def _head1_body(rst_ref, rd_ref, wo_ref, qres_ref, z_ref, s_ref, q_ref):
    rd = rd_ref[...]
    rscale = jnp.concatenate(
        [jnp.broadcast_to(rd[:, h:h + 1], (_NBLK, OUT_DIM))
         for h in range(HEADS)], axis=1)
    z = jnp.dot(rst_ref[...] * rscale, wo_ref[...],
                preferred_element_type=jnp.float32) + qres_ref[...]
    z_ref[...] = z

    @pl.when(pl.program_id(0) == 0)
    def _():
        s_ref[...] = jnp.zeros_like(s_ref)
        q_ref[...] = jnp.zeros_like(q_ref)

    s_ref[...] += jnp.sum(z, axis=0, keepdims=True)
    q_ref[...] += jnp.sum(z * z, axis=0, keepdims=True)


def _head1(rst, rdenN, Wo, Qres):
    nb = N // _NBLK
    blk = lambda i: (i, 0)
    acc = pl.BlockSpec((1, C), lambda i: (0, 0))
    return pl.pallas_call(
        _head1_body,
        grid=(nb,),
        in_specs=[pl.BlockSpec((_NBLK, C), blk),
                  pl.BlockSpec((_NBLK, 128), blk),
                  pl.BlockSpec((C, C), lambda i: (0, 0)),
                  pl.BlockSpec((_NBLK, C), blk)],
        out_specs=[pl.BlockSpec((_NBLK, C), blk), acc, acc],
        out_shape=[jax.ShapeDtypeStruct((N, C), jnp.float32),
                   jax.ShapeDtypeStruct((1, C), jnp.float32),
                   jax.ShapeDtypeStruct((1, C), jnp.float32)],
    )(rst, rdenN, Wo, Qres)


def _head2_body(z_ref, a1_ref, b1_ref, w1_ref, bf1_ref, w2_ref, bf2_ref,
                y_ref, s_ref, q_ref):
    zn = z_ref[...] * a1_ref[...] + b1_ref[...]
    h = jnp.maximum(jnp.dot(zn, w1_ref[...],
                            preferred_element_type=jnp.float32)
                    + bf1_ref[...], 0.0)
    y = jnp.dot(h, w2_ref[...],
                preferred_element_type=jnp.float32) + bf2_ref[...] + zn
    y_ref[...] = y

    @pl.when(pl.program_id(0) == 0)
    def _():
        s_ref[...] = jnp.zeros_like(s_ref)
        q_ref[...] = jnp.zeros_like(q_ref)

    s_ref[...] += jnp.sum(y, axis=0, keepdims=True)
    q_ref[...] += jnp.sum(y * y, axis=0, keepdims=True)


def _head2(Z, a1, b1, W1, bf1, W2, bf2):
    nb = N // _NBLK
    blk = lambda i: (i, 0)
    acc = pl.BlockSpec((1, C), lambda i: (0, 0))
    one = lambda shape: pl.BlockSpec(shape, lambda i: (0, 0))
    return pl.pallas_call(
        _head2_body,
        grid=(nb,),
        in_specs=[pl.BlockSpec((_NBLK, C), blk),
                  one((1, C)), one((1, C)),
                  one((C, 2 * C)), one((1, 2 * C)),
                  one((2 * C, C)), one((1, C))],
        out_specs=[pl.BlockSpec((_NBLK, C), blk), acc, acc],
        out_shape=[jax.ShapeDtypeStruct((N, C), jnp.float32),
                   jax.ShapeDtypeStruct((1, C), jnp.float32),
                   jax.ShapeDtypeStruct((1, C), jnp.float32)],
    )(Z, a1, b1, W1, bf1, W2, bf2)


def _bn2_body(y_ref, a2_ref, b2_ref, out_ref):
    out_ref[...] = y_ref[...] * a2_ref[...] + b2_ref[...]


def _bn2(Y, a2, b2):
    nb = N // _NBLK
    blk = lambda i: (i, 0)
    return pl.pallas_call(
        _bn2_body,
        grid=(nb,),
        in_specs=[pl.BlockSpec((_NBLK, C), blk),
                  pl.BlockSpec((1, C), lambda i: (0, 0)),
                  pl.BlockSpec((1, C), lambda i: (0, 0))],
        out_specs=pl.BlockSpec((_NBLK, C), blk),
        out_shape=jax.ShapeDtypeStruct((N, C), jnp.float32),
    )(Y, a2, b2)


def _sc_gather_body(Q_hbm, K_hbm, dstR_hbm, srcR_hbm, Qg_hbm, Kg_hbm,
                    ibuf, buf):
    c = lax.axis_index("c")
    s = lax.axis_index("s")
    w = s * NC + c
    lo = w * 39 + jnp.minimum(w, 2)
    cnt = 39 + jnp.where(w < 2, 1, 0)

    @pl.loop(0, cnt)
    def _(i):
        r = lo + i
        base = r * CH
        pltpu.sync_copy(dstR_hbm.at[r], ibuf)
        pltpu.sync_copy(Q_hbm.at[ibuf.at[0]], buf)
        pltpu.sync_copy(buf, Qg_hbm.at[pl.ds(base, CH)])
        pltpu.sync_copy(srcR_hbm.at[r], ibuf)
        pltpu.sync_copy(K_hbm.at[ibuf.at[0]], buf)
        pltpu.sync_copy(buf, Kg_hbm.at[pl.ds(base, CH)])


def _sc_gather(Q, K, dstR, srcR):
    f = pl.kernel(
        _sc_gather_body,
        out_type=(jax.ShapeDtypeStruct((E_PAD, C), jnp.float32),
                  jax.ShapeDtypeStruct((E_PAD, C), jnp.float32)),
        mesh=_MESH,
        scratch_types=[pltpu.VMEM((1, CH), jnp.int32),
                       pltpu.VMEM((CH, C), jnp.float32)],
    )
    return f(Q, K, dstR, srcR)


def _sc_denom_body(ex16_hbm, dstR_hbm, z128_hbm, dpart_hbm,
                   ibuf, exbuf, padbuf, acc):
    c = lax.axis_index("c")
    s = lax.axis_index("s")
    row0 = s * ROWS_PER_TILE
    pltpu.sync_copy(z128_hbm, acc.at[pl.ds(row0, ROWS_PER_TILE)])
    pltpu.sync_copy(z128_hbm.at[pl.ds(0, CH)], padbuf)
    plsc.subcore_barrier()

    lo = c * 625 + s * 39 + jnp.minimum(s, 1)
    cnt = 39 + jnp.where(s < 1, 1, 0)

    @pl.loop(0, cnt)
    def _(i):
        r = lo + i
        pltpu.sync_copy(dstR_hbm.at[r], ibuf)
        pltpu.sync_copy(ex16_hbm.at[pl.ds(r * CH, CH)], exbuf)

        @pl.loop(0, CH)
        def _(j):
            padbuf[j, pl.ds(0, 16)] = exbuf[j, :]

        pltpu.sync_copy(padbuf, acc.at[ibuf.at[0]], add=True)

    plsc.subcore_barrier()
    pltpu.sync_copy(acc.at[pl.ds(row0, ROWS_PER_TILE)],
                    dpart_hbm.at[c, pl.ds(row0, ROWS_PER_TILE)])


def _sc_denom(ex16, dstR, z128):
    f = pl.kernel(
        _sc_denom_body,
        out_type=jax.ShapeDtypeStruct((NC, NP, 128), jnp.float32),
        mesh=_MESH,
        scratch_types=[pltpu.VMEM((1, CH), jnp.int32),
                       pltpu.VMEM((CH, 16), jnp.float32),
                       pltpu.VMEM((CH, 128), jnp.float32),
                       pltpu.VMEM_SHARED((NP, 128), jnp.float32)],
    )
    return f(ex16, dstR, z128)


def _sc_msg_body(VR_hbm, exX_hbm, dstR_hbm, srcR_hbm, z128_hbm,
                 rst_hbm, dbuf, sbuf, i2buf, axbuf, vbuf, acc):
    c = lax.axis_index("c")
    s = lax.axis_index("s")
    row0 = s * ROWS_PER_TILE
    lo = s * 78 + jnp.minimum(s, 2)
    cnt = 78 + jnp.where(s < 2, 1, 0)

    for p in range(2):
        g = c * 2 + p
        pltpu.sync_copy(z128_hbm, acc.at[pl.ds(row0, ROWS_PER_TILE)])
        plsc.subcore_barrier()

        @pl.loop(0, cnt)
        def _(i):
            r = lo + i
            base = r * CH
            pltpu.sync_copy(dstR_hbm.at[r], dbuf)
            pltpu.sync_copy(srcR_hbm.at[r], sbuf)
            pltpu.sync_copy(exX_hbm.at[pl.ds(base, CH),
                                       pl.ds(g * 128, 128)], axbuf)
            for j in range(CH // 16):
                sv = sbuf[0, pl.ds(j * 16, 16)]
                i2buf[0, pl.ds(j * 16, 16)] = sv * 4 + g
            pltpu.sync_copy(VR_hbm.at[i2buf.at[0]], vbuf)

            @pl.loop(0, CH)
            def _(j):
                for t in range(8):
                    sl = pl.ds(t * 16, 16)
                    vbuf[j, sl] = vbuf[j, sl] * axbuf[j, sl]

            pltpu.sync_copy(vbuf, acc.at[dbuf.at[0]], add=True)

        plsc.subcore_barrier()
        pltpu.sync_copy(acc.at[pl.ds(row0, ROWS_PER_TILE)],
                        rst_hbm.at[pl.ds(row0, ROWS_PER_TILE),
                                   pl.ds(g * 128, 128)])
        plsc.subcore_barrier()


def _sc_msg(VR, exX, dstR, srcR, z128):
    f = pl.kernel(
        _sc_msg_body,
        out_type=jax.ShapeDtypeStruct((NP, C), jnp.float32),
        mesh=_MESH,
        scratch_types=[pltpu.VMEM((1, CH), jnp.int32),
                       pltpu.VMEM((1, CH), jnp.int32),
                       pltpu.VMEM((1, CH), jnp.int32),
                       pltpu.VMEM((CH, 128), jnp.float32),
                       pltpu.VMEM((CH, 128), jnp.float32),
                       pltpu.VMEM_SHARED((NP, 128), jnp.float32)],
    )
    return f(VR, exX, dstR, srcR, z128)


def kernel(q, k, v, edge_feat, edge_index, Wq, Wk, Wv, We, be, Wo, Wres,
           W1, bf1, W2, bf2, gamma1, beta1, gamma2, beta2):
    src = edge_index[0].astype(jnp.int32)
    dst = edge_index[1].astype(jnp.int32)
    pad = jnp.zeros((E_PAD - E,), jnp.int32)
    dstR = jnp.concatenate([dst, pad]).reshape(NCH, 1, CH)
    srcR = jnp.concatenate([src, pad]).reshape(NCH, 1, CH)
    efP = jnp.concatenate(
        [edge_feat, jnp.zeros((E_PAD - E, 16), jnp.float32)], axis=0)

    WeP = We
    WeP = jnp.concatenate([We, jnp.zeros((16, 8), jnp.float32)], axis=1)
    beP = jnp.concatenate([be, jnp.full((8,), -1e30, jnp.float32)])
    beP = beP.reshape(1, 16)

    m_np = np.zeros((C, 16), np.float32)
    for h in range(HEADS):
        m_np[h * OUT_DIM:(h + 1) * OUT_DIM, h] = 1.0
    M = jnp.asarray(m_np)

    z128 = jnp.zeros((ROWS_PER_TILE, 128), jnp.float32)

    Q, K, V, Qres = _proj(q, k, v, Wq, Wk, Wv, Wres)
    VR = V.reshape(N * 4, 128)

    Qg, Kg = _sc_gather(Q, K, dstR, srcR)
    ex16, exX = _scores(Qg, Kg, efP, WeP, beP, M)

    dpart = _sc_denom(ex16, dstR, z128)
    rden = _rden(dpart)

    exX, rden = lax.optimization_barrier((exX, rden))

    rst = _sc_msg(VR, exX, dstR, srcR, z128)
    rst = rst[:N]
    rdenN = rden[:N]

    Z, s1, q1 = _head1(rst, rdenN, Wo, Qres)
    mean1 = s1 / N
    var1 = q1 / N - mean1 * mean1
    a1 = gamma1 / jnp.sqrt(var1 + 1e-5)
    b1 = beta1 - mean1 * a1

    Y, s2, q2 = _head2(Z, a1, b1, W1, bf1.reshape(1, -1), W2,
                       bf2.reshape(1, -1))
    mean2 = s2 / N
    var2 = q2 / N - mean2 * mean2
    a2 = gamma2 / jnp.sqrt(var2 + 1e-5)
    b2 = beta2 - mean2 * a2

    return _bn2(Y, a2, b2)


# speedup vs baseline: 1.5920x; 1.1115x over previous
import dataclasses
import functools

import jax
import jax.numpy as jnp
import numpy as np
from jax import lax
from jax.experimental import pallas as pl
from jax.experimental.pallas import tpu as pltpu
from jax.experimental.pallas import tpu_sc as plsc

N = 10000
E = 160000
IN_DIM = 256
OUT_DIM = 64
HEADS = 8
C = OUT_DIM * HEADS

CH = 128
E_PAD = 160000
NCH = E_PAD // CH
NC, NS = 2, 16
NP = 10240
ROWS_PER_TILE = NP // NS

_MESH = plsc.VectorSubcoreMesh(core_axis_name="c", subcore_axis_name="s")

_SC_CP = pltpu.CompilerParams()
if "needs_layout_passes" in pltpu.CompilerParams.__dataclass_fields__:
    _SC_CP = dataclasses.replace(_SC_CP, needs_layout_passes=False)

_NBLK = 400
_EBLK = 2000


def _proj_body(q_ref, k_ref, v_ref, wq_ref, wk_ref, wv_ref, wr_ref,
               Q_ref, K_ref, V_ref, R_ref):
    Q_ref[...] = jnp.dot(q_ref[...], wq_ref[...],
                         preferred_element_type=jnp.float32)
    K_ref[...] = jnp.dot(k_ref[...], wk_ref[...],
                         preferred_element_type=jnp.float32)
    V_ref[...] = jnp.dot(v_ref[...], wv_ref[...],
                         preferred_element_type=jnp.float32)
    R_ref[...] = jnp.dot(q_ref[...], wr_ref[...],
                         preferred_element_type=jnp.float32)


def _proj(q, k, v, Wq, Wk, Wv, Wres):
    nb = N // _NBLK
    blk = lambda i: (i, 0)
    w_spec = pl.BlockSpec((IN_DIM, C), lambda i: (0, 0))
    return pl.pallas_call(
        _proj_body,
        grid=(nb,),
        in_specs=[pl.BlockSpec((_NBLK, IN_DIM), blk)] * 3 + [w_spec] * 4,
        out_specs=[pl.BlockSpec((_NBLK, C), blk)] * 4,
        out_shape=[jax.ShapeDtypeStruct((N, C), jnp.float32)] * 4,
    )(q, k, v, Wq, Wk, Wv, Wres)


def _scores_body(qg_ref, kg_ref, ef_ref, wep_ref, bep_ref, m_ref, ex16_ref):
    prod = qg_ref[...] * kg_ref[...]
    s = jnp.dot(prod, m_ref[...], preferred_element_type=jnp.float32)
    s = s * (1.0 / float(OUT_DIM) ** 0.5)
    s = s + jnp.dot(ef_ref[...], wep_ref[...],
                    preferred_element_type=jnp.float32) + bep_ref[...]
    row = (pl.program_id(0) * _EBLK
           + lax.broadcasted_iota(jnp.int32, (_EBLK, 1), 0))
    s = jnp.where(row < E, s, -1e30)
    ex16_ref[...] = jnp.exp(s)


def _scores(Qg, Kg, ef, WeP, beP, M):
    nb = E_PAD // _EBLK
    blk = lambda i: (i, 0)
    return pl.pallas_call(
        _scores_body,
        grid=(nb,),
        in_specs=[pl.BlockSpec((_EBLK, C), blk),
                  pl.BlockSpec((_EBLK, C), blk),
                  pl.BlockSpec((_EBLK, 16), blk),
                  pl.BlockSpec((16, 16), lambda i: (0, 0)),
                  pl.BlockSpec((1, 16), lambda i: (0, 0)),
                  pl.BlockSpec((C, 16), lambda i: (0, 0))],
        out_specs=pl.BlockSpec((_EBLK, 16), blk),
        out_shape=jax.ShapeDtypeStruct((E_PAD, 16), jnp.float32),
    )(Qg, Kg, ef, WeP, beP, M)


def _rden_body(dp_ref, out_ref):
    den = dp_ref[0] + dp_ref[1]
    out_ref[...] = 1.0 / jnp.maximum(den, 1e-30)


def _rden(dpart):
    return pl.pallas_call(
        _rden_body,
        out_shape=jax.ShapeDtypeStruct((NP, 128), jnp.float32),
    )(dpart)


def _head1_body(rst_ref, rd_ref, wo_ref, qres_ref, z_ref, s_ref, q_ref):
    rd = rd_ref[...]
    rscale = jnp.concatenate(
        [jnp.broadcast_to(rd[:, h:h + 1], (_NBLK, OUT_DIM))
         for h in range(HEADS)], axis=1)
    z = jnp.dot(rst_ref[...] * rscale, wo_ref[...],
                preferred_element_type=jnp.float32) + qres_ref[...]
    z_ref[...] = z

    @pl.when(pl.program_id(0) == 0)
    def _():
        s_ref[...] = jnp.zeros_like(s_ref)
        q_ref[...] = jnp.zeros_like(q_ref)

    s_ref[...] += jnp.sum(z, axis=0, keepdims=True)
    q_ref[...] += jnp.sum(z * z, axis=0, keepdims=True)


def _head1(rst, rdenN, Wo, Qres):
    nb = N // _NBLK
    blk = lambda i: (i, 0)
    acc = pl.BlockSpec((1, C), lambda i: (0, 0))
    return pl.pallas_call(
        _head1_body,
        grid=(nb,),
        in_specs=[pl.BlockSpec((_NBLK, C), blk),
                  pl.BlockSpec((_NBLK, 128), blk),
                  pl.BlockSpec((C, C), lambda i: (0, 0)),
                  pl.BlockSpec((_NBLK, C), blk)],
        out_specs=[pl.BlockSpec((_NBLK, C), blk), acc, acc],
        out_shape=[jax.ShapeDtypeStruct((N, C), jnp.float32),
                   jax.ShapeDtypeStruct((1, C), jnp.float32),
                   jax.ShapeDtypeStruct((1, C), jnp.float32)],
    )(rst, rdenN, Wo, Qres)


def _head2_body(z_ref, a1_ref, b1_ref, w1_ref, bf1_ref, w2_ref, bf2_ref,
                y_ref, s_ref, q_ref):
    zn = z_ref[...] * a1_ref[...] + b1_ref[...]
    h = jnp.maximum(jnp.dot(zn, w1_ref[...],
                            preferred_element_type=jnp.float32)
                    + bf1_ref[...], 0.0)
    y = jnp.dot(h, w2_ref[...],
                preferred_element_type=jnp.float32) + bf2_ref[...] + zn
    y_ref[...] = y

    @pl.when(pl.program_id(0) == 0)
    def _():
        s_ref[...] = jnp.zeros_like(s_ref)
        q_ref[...] = jnp.zeros_like(q_ref)

    s_ref[...] += jnp.sum(y, axis=0, keepdims=True)
    q_ref[...] += jnp.sum(y * y, axis=0, keepdims=True)


def _head2(Z, a1, b1, W1, bf1, W2, bf2):
    nb = N // _NBLK
    blk = lambda i: (i, 0)
    acc = pl.BlockSpec((1, C), lambda i: (0, 0))
    one = lambda shape: pl.BlockSpec(shape, lambda i: (0, 0))
    return pl.pallas_call(
        _head2_body,
        grid=(nb,),
        in_specs=[pl.BlockSpec((_NBLK, C), blk),
                  one((1, C)), one((1, C)),
                  one((C, 2 * C)), one((1, 2 * C)),
                  one((2 * C, C)), one((1, C))],
        out_specs=[pl.BlockSpec((_NBLK, C), blk), acc, acc],
        out_shape=[jax.ShapeDtypeStruct((N, C), jnp.float32),
                   jax.ShapeDtypeStruct((1, C), jnp.float32),
                   jax.ShapeDtypeStruct((1, C), jnp.float32)],
    )(Z, a1, b1, W1, bf1, W2, bf2)


def _bn2_body(y_ref, a2_ref, b2_ref, out_ref):
    out_ref[...] = y_ref[...] * a2_ref[...] + b2_ref[...]


def _bn2(Y, a2, b2):
    nb = N // _NBLK
    blk = lambda i: (i, 0)
    return pl.pallas_call(
        _bn2_body,
        grid=(nb,),
        in_specs=[pl.BlockSpec((_NBLK, C), blk),
                  pl.BlockSpec((1, C), lambda i: (0, 0)),
                  pl.BlockSpec((1, C), lambda i: (0, 0))],
        out_specs=pl.BlockSpec((_NBLK, C), blk),
        out_shape=jax.ShapeDtypeStruct((N, C), jnp.float32),
    )(Y, a2, b2)


def _sc_gather_body(Q_hbm, K_hbm, dstR_hbm, srcR_hbm, Qg_hbm, Kg_hbm,
                    ibuf, buf):
    c = lax.axis_index("c")
    s = lax.axis_index("s")
    w = s * NC + c
    lo = w * 39 + jnp.minimum(w, 2)
    cnt = 39 + jnp.where(w < 2, 1, 0)

    @pl.loop(0, cnt)
    def _(i):
        r = lo + i
        base = r * CH
        pltpu.sync_copy(dstR_hbm.at[r], ibuf)
        pltpu.sync_copy(Q_hbm.at[ibuf.at[0]], buf)
        pltpu.sync_copy(buf, Qg_hbm.at[pl.ds(base, CH)])
        pltpu.sync_copy(srcR_hbm.at[r], ibuf)
        pltpu.sync_copy(K_hbm.at[ibuf.at[0]], buf)
        pltpu.sync_copy(buf, Kg_hbm.at[pl.ds(base, CH)])


def _sc_gather(Q, K, dstR, srcR):
    f = pl.kernel(
        _sc_gather_body,
        out_type=(jax.ShapeDtypeStruct((E_PAD, C), jnp.float32),
                  jax.ShapeDtypeStruct((E_PAD, C), jnp.float32)),
        mesh=_MESH,
        scratch_types=[pltpu.VMEM((1, CH), jnp.int32),
                       pltpu.VMEM((CH, C), jnp.float32)],
    )
    return f(Q, K, dstR, srcR)


def _sc_denom_body(ex16_hbm, dstR_hbm, z128_hbm, dpart_hbm,
                   ibuf, exbuf, padbuf, acc):
    c = lax.axis_index("c")
    s = lax.axis_index("s")
    row0 = s * ROWS_PER_TILE
    pltpu.sync_copy(z128_hbm, acc.at[pl.ds(row0, ROWS_PER_TILE)])
    pltpu.sync_copy(z128_hbm.at[pl.ds(0, CH)], padbuf)
    plsc.subcore_barrier()

    lo = c * 625 + s * 39 + jnp.minimum(s, 1)
    cnt = 39 + jnp.where(s < 1, 1, 0)

    @pl.loop(0, cnt)
    def _(i):
        r = lo + i
        pltpu.sync_copy(dstR_hbm.at[r], ibuf)
        pltpu.sync_copy(ex16_hbm.at[pl.ds(r * CH, CH)], exbuf)

        @pl.loop(0, CH)
        def _(j):
            padbuf[j, pl.ds(0, 16)] = exbuf[j, :]

        pltpu.sync_copy(padbuf, acc.at[ibuf.at[0]], add=True)

    plsc.subcore_barrier()
    pltpu.sync_copy(acc.at[pl.ds(row0, ROWS_PER_TILE)],
                    dpart_hbm.at[c, pl.ds(row0, ROWS_PER_TILE)])


def _sc_denom(ex16, dstR, z128):
    f = pl.kernel(
        _sc_denom_body,
        out_type=jax.ShapeDtypeStruct((NC, NP, 128), jnp.float32),
        mesh=_MESH,
        scratch_types=[pltpu.VMEM((1, CH), jnp.int32),
                       pltpu.VMEM((CH, 16), jnp.float32),
                       pltpu.VMEM((CH, 128), jnp.float32),
                       pltpu.VMEM_SHARED((NP, 128), jnp.float32)],
    )
    return f(ex16, dstR, z128)


_GDN = lax.GatherDimensionNumbers(
    offset_dims=(), collapsed_slice_dims=(0,), start_index_map=(0,))


def _splat(row, col):
    idx = jnp.full((16, 1), col, jnp.int32)
    return lax.gather(row, idx, _GDN, (1,),
                      mode=lax.GatherScatterMode.PROMISE_IN_BOUNDS)


def _sc_msg_body(VR_hbm, ex16_hbm, dstR_hbm, srcR_hbm, z128_hbm,
                 rst_hbm, dbuf, sbuf, i2buf, exbuf, vbuf, acc):
    c = lax.axis_index("c")
    s = lax.axis_index("s")
    row0 = s * ROWS_PER_TILE
    lo = s * 78 + jnp.minimum(s, 2)
    cnt = 78 + jnp.where(s < 2, 1, 0)

    for p in range(2):
        g = c * 2 + p
        pltpu.sync_copy(z128_hbm, acc.at[pl.ds(row0, ROWS_PER_TILE)])
        plsc.subcore_barrier()

        @pl.loop(0, cnt)
        def _(i):
            r = lo + i
            base = r * CH
            pltpu.sync_copy(dstR_hbm.at[r], dbuf)
            pltpu.sync_copy(srcR_hbm.at[r], sbuf)
            pltpu.sync_copy(ex16_hbm.at[pl.ds(base, CH)], exbuf)
            for j in range(CH // 16):
                sv = sbuf[0, pl.ds(j * 16, 16)]
                i2buf[0, pl.ds(j * 16, 16)] = sv * 4 + g
            pltpu.sync_copy(VR_hbm.at[i2buf.at[0]], vbuf)

            @pl.loop(0, CH)
            def _(j):
                row = exbuf[j, :]
                a0 = _splat(row, 2 * g)
                a1 = _splat(row, 2 * g + 1)
                for t in range(4):
                    sl = pl.ds(t * 16, 16)
                    vbuf[j, sl] = vbuf[j, sl] * a0
                for t in range(4, 8):
                    sl = pl.ds(t * 16, 16)
                    vbuf[j, sl] = vbuf[j, sl] * a1

            pltpu.sync_copy(vbuf, acc.at[dbuf.at[0]], add=True)

        plsc.subcore_barrier()
        pltpu.sync_copy(acc.at[pl.ds(row0, ROWS_PER_TILE)],
                        rst_hbm.at[pl.ds(row0, ROWS_PER_TILE),
                                   pl.ds(g * 128, 128)])
        plsc.subcore_barrier()


def _sc_msg(VR, ex16, dstR, srcR, z128):
    f = pl.kernel(
        _sc_msg_body,
        out_type=jax.ShapeDtypeStruct((NP, C), jnp.float32),
        mesh=_MESH,
        scratch_types=[pltpu.VMEM((1, CH), jnp.int32),
                       pltpu.VMEM((1, CH), jnp.int32),
                       pltpu.VMEM((1, CH), jnp.int32),
                       pltpu.VMEM((CH, 16), jnp.float32),
                       pltpu.VMEM((CH, 128), jnp.float32),
                       pltpu.VMEM_SHARED((NP, 128), jnp.float32)],
    )
    return f(VR, ex16, dstR, srcR, z128)


def kernel(q, k, v, edge_feat, edge_index, Wq, Wk, Wv, We, be, Wo, Wres,
           W1, bf1, W2, bf2, gamma1, beta1, gamma2, beta2):
    src = edge_index[0].astype(jnp.int32)
    dst = edge_index[1].astype(jnp.int32)
    pad = jnp.zeros((E_PAD - E,), jnp.int32)
    dstR = jnp.concatenate([dst, pad]).reshape(NCH, 1, CH)
    srcR = jnp.concatenate([src, pad]).reshape(NCH, 1, CH)
    efP = jnp.concatenate(
        [edge_feat, jnp.zeros((E_PAD - E, 16), jnp.float32)], axis=0)

    WeP = We
    WeP = jnp.concatenate([We, jnp.zeros((16, 8), jnp.float32)], axis=1)
    beP = jnp.concatenate([be, jnp.full((8,), -1e30, jnp.float32)])
    beP = beP.reshape(1, 16)

    m_np = np.zeros((C, 16), np.float32)
    for h in range(HEADS):
        m_np[h * OUT_DIM:(h + 1) * OUT_DIM, h] = 1.0
    M = jnp.asarray(m_np)

    z128 = jnp.zeros((ROWS_PER_TILE, 128), jnp.float32)

    Q, K, V, Qres = _proj(q, k, v, Wq, Wk, Wv, Wres)
    VR = V.reshape(N * 4, 128)

    Qg, Kg = _sc_gather(Q, K, dstR, srcR)
    ex16 = _scores(Qg, Kg, efP, WeP, beP, M)

    dpart = _sc_denom(ex16, dstR, z128)
    rden = _rden(dpart)

    rst = _sc_msg(VR, ex16, dstR, srcR, z128)
    rst = rst[:N]
    rdenN = rden[:N]

    Z, s1, q1 = _head1(rst, rdenN, Wo, Qres)
    mean1 = s1 / N
    var1 = q1 / N - mean1 * mean1
    a1 = gamma1 / jnp.sqrt(var1 + 1e-5)
    b1 = beta1 - mean1 * a1

    Y, s2, q2 = _head2(Z, a1, b1, W1, bf1.reshape(1, -1), W2,
                       bf2.reshape(1, -1))
    mean2 = s2 / N
    var2 = q2 / N - mean2 * mean2
    a2 = gamma2 / jnp.sqrt(var2 + 1e-5)
    b2 = beta2 - mean2 * a2

    return _bn2(Y, a2, b2)


# speedup vs baseline: 1.8353x; 1.1528x over previous
import dataclasses
import functools

import jax
import jax.numpy as jnp
import numpy as np
from jax import lax
from jax.experimental import pallas as pl
from jax.experimental.pallas import tpu as pltpu
from jax.experimental.pallas import tpu_sc as plsc

N = 10000
E = 160000
IN_DIM = 256
OUT_DIM = 64
HEADS = 8
C = OUT_DIM * HEADS

CH = 128
E_PAD = 160000
NCH = E_PAD // CH
NC, NS = 2, 16
NP = 10112
ROWS_PER_TILE = NP // NS

_MESH = plsc.VectorSubcoreMesh(core_axis_name="c", subcore_axis_name="s")

_SC_CP = pltpu.CompilerParams()
if "needs_layout_passes" in pltpu.CompilerParams.__dataclass_fields__:
    _SC_CP = dataclasses.replace(_SC_CP, needs_layout_passes=False)

_NBLK = 400
_EBLK = 2000


def _proj_body(q_ref, k_ref, v_ref, wq_ref, wk_ref, wv_ref, wr_ref,
               Q_ref, K_ref, V_ref, R_ref):
    Q_ref[...] = jnp.dot(q_ref[...], wq_ref[...],
                         preferred_element_type=jnp.float32)
    K_ref[...] = jnp.dot(k_ref[...], wk_ref[...],
                         preferred_element_type=jnp.float32)
    V_ref[...] = jnp.dot(v_ref[...], wv_ref[...],
                         preferred_element_type=jnp.float32)
    R_ref[...] = jnp.dot(q_ref[...], wr_ref[...],
                         preferred_element_type=jnp.float32)


def _proj(q, k, v, Wq, Wk, Wv, Wres):
    nb = N // _NBLK
    blk = lambda i: (i, 0)
    w_spec = pl.BlockSpec((IN_DIM, C), lambda i: (0, 0))
    return pl.pallas_call(
        _proj_body,
        grid=(nb,),
        in_specs=[pl.BlockSpec((_NBLK, IN_DIM), blk)] * 3 + [w_spec] * 4,
        out_specs=[pl.BlockSpec((_NBLK, C), blk)] * 4,
        out_shape=[jax.ShapeDtypeStruct((N, C), jnp.float32)] * 4,
    )(q, k, v, Wq, Wk, Wv, Wres)


def _scores_body(qg_ref, kg_ref, ef_ref, wep_ref, bep_ref, m_ref, ex16_ref):
    prod = qg_ref[...] * kg_ref[...]
    s = jnp.dot(prod, m_ref[...], preferred_element_type=jnp.float32)
    s = s * (1.0 / float(OUT_DIM) ** 0.5)
    s = s + jnp.dot(ef_ref[...], wep_ref[...],
                    preferred_element_type=jnp.float32) + bep_ref[...]
    row = (pl.program_id(0) * _EBLK
           + lax.broadcasted_iota(jnp.int32, (_EBLK, 1), 0))
    s = jnp.where(row < E, s, -1e30)
    ex16_ref[...] = jnp.exp(s)


def _scores(Qg, Kg, ef, WeP, beP, M):
    nb = E_PAD // _EBLK
    blk = lambda i: (i, 0)
    return pl.pallas_call(
        _scores_body,
        grid=(nb,),
        in_specs=[pl.BlockSpec((_EBLK, C), blk),
                  pl.BlockSpec((_EBLK, C), blk),
                  pl.BlockSpec((_EBLK, 16), blk),
                  pl.BlockSpec((16, 16), lambda i: (0, 0)),
                  pl.BlockSpec((1, 16), lambda i: (0, 0)),
                  pl.BlockSpec((C, 16), lambda i: (0, 0))],
        out_specs=pl.BlockSpec((_EBLK, 16), blk),
        out_shape=jax.ShapeDtypeStruct((E_PAD, 16), jnp.float32),
    )(Qg, Kg, ef, WeP, beP, M)


def _rden_body(dp_ref, out_ref):
    den = dp_ref[0] + dp_ref[1]
    out_ref[...] = 1.0 / jnp.maximum(den, 1e-30)


def _rden(dpart):
    return pl.pallas_call(
        _rden_body,
        out_shape=jax.ShapeDtypeStruct((NP, 128), jnp.float32),
    )(dpart)


def _head1_body(rst_ref, rd_ref, wo_ref, qres_ref, z_ref, s_ref, q_ref):
    rd = rd_ref[...]
    rscale = jnp.concatenate(
        [jnp.broadcast_to(rd[:, h:h + 1], (_NBLK, OUT_DIM))
         for h in range(HEADS)], axis=1)
    z = jnp.dot(rst_ref[...] * rscale, wo_ref[...],
                preferred_element_type=jnp.float32) + qres_ref[...]
    z_ref[...] = z

    @pl.when(pl.program_id(0) == 0)
    def _():
        s_ref[...] = jnp.zeros_like(s_ref)
        q_ref[...] = jnp.zeros_like(q_ref)

    s_ref[...] += jnp.sum(z, axis=0, keepdims=True)
    q_ref[...] += jnp.sum(z * z, axis=0, keepdims=True)


def _head1(rst, rdenN, Wo, Qres):
    nb = N // _NBLK
    blk = lambda i: (i, 0)
    acc = pl.BlockSpec((1, C), lambda i: (0, 0))
    return pl.pallas_call(
        _head1_body,
        grid=(nb,),
        in_specs=[pl.BlockSpec((_NBLK, C), blk),
                  pl.BlockSpec((_NBLK, 128), blk),
                  pl.BlockSpec((C, C), lambda i: (0, 0)),
                  pl.BlockSpec((_NBLK, C), blk)],
        out_specs=[pl.BlockSpec((_NBLK, C), blk), acc, acc],
        out_shape=[jax.ShapeDtypeStruct((N, C), jnp.float32),
                   jax.ShapeDtypeStruct((1, C), jnp.float32),
                   jax.ShapeDtypeStruct((1, C), jnp.float32)],
    )(rst, rdenN, Wo, Qres)


def _head2_body(z_ref, a1_ref, b1_ref, w1_ref, bf1_ref, w2_ref, bf2_ref,
                y_ref, s_ref, q_ref):
    zn = z_ref[...] * a1_ref[...] + b1_ref[...]
    h = jnp.maximum(jnp.dot(zn, w1_ref[...],
                            preferred_element_type=jnp.float32)
                    + bf1_ref[...], 0.0)
    y = jnp.dot(h, w2_ref[...],
                preferred_element_type=jnp.float32) + bf2_ref[...] + zn
    y_ref[...] = y

    @pl.when(pl.program_id(0) == 0)
    def _():
        s_ref[...] = jnp.zeros_like(s_ref)
        q_ref[...] = jnp.zeros_like(q_ref)

    s_ref[...] += jnp.sum(y, axis=0, keepdims=True)
    q_ref[...] += jnp.sum(y * y, axis=0, keepdims=True)


def _head2(Z, a1, b1, W1, bf1, W2, bf2):
    nb = N // _NBLK
    blk = lambda i: (i, 0)
    acc = pl.BlockSpec((1, C), lambda i: (0, 0))
    one = lambda shape: pl.BlockSpec(shape, lambda i: (0, 0))
    return pl.pallas_call(
        _head2_body,
        grid=(nb,),
        in_specs=[pl.BlockSpec((_NBLK, C), blk),
                  one((1, C)), one((1, C)),
                  one((C, 2 * C)), one((1, 2 * C)),
                  one((2 * C, C)), one((1, C))],
        out_specs=[pl.BlockSpec((_NBLK, C), blk), acc, acc],
        out_shape=[jax.ShapeDtypeStruct((N, C), jnp.float32),
                   jax.ShapeDtypeStruct((1, C), jnp.float32),
                   jax.ShapeDtypeStruct((1, C), jnp.float32)],
    )(Z, a1, b1, W1, bf1, W2, bf2)


def _bn2_body(y_ref, a2_ref, b2_ref, out_ref):
    out_ref[...] = y_ref[...] * a2_ref[...] + b2_ref[...]


def _bn2(Y, a2, b2):
    nb = N // _NBLK
    blk = lambda i: (i, 0)
    return pl.pallas_call(
        _bn2_body,
        grid=(nb,),
        in_specs=[pl.BlockSpec((_NBLK, C), blk),
                  pl.BlockSpec((1, C), lambda i: (0, 0)),
                  pl.BlockSpec((1, C), lambda i: (0, 0))],
        out_specs=pl.BlockSpec((_NBLK, C), blk),
        out_shape=jax.ShapeDtypeStruct((N, C), jnp.float32),
    )(Y, a2, b2)


def _sc_gather_body(Q_hbm, K_hbm, dstR_hbm, srcR_hbm, Qg_hbm, Kg_hbm,
                    ibuf, buf):
    c = lax.axis_index("c")
    s = lax.axis_index("s")
    w = s * NC + c
    lo = w * 39 + jnp.minimum(w, 2)
    cnt = 39 + jnp.where(w < 2, 1, 0)

    @pl.loop(0, cnt)
    def _(i):
        r = lo + i
        base = r * CH
        pltpu.sync_copy(dstR_hbm.at[r], ibuf)
        pltpu.sync_copy(Q_hbm.at[ibuf.at[0]], buf)
        pltpu.sync_copy(buf, Qg_hbm.at[pl.ds(base, CH)])
        pltpu.sync_copy(srcR_hbm.at[r], ibuf)
        pltpu.sync_copy(K_hbm.at[ibuf.at[0]], buf)
        pltpu.sync_copy(buf, Kg_hbm.at[pl.ds(base, CH)])


def _sc_gather(Q, K, dstR, srcR):
    f = pl.kernel(
        _sc_gather_body,
        out_type=(jax.ShapeDtypeStruct((E_PAD, C), jnp.float32),
                  jax.ShapeDtypeStruct((E_PAD, C), jnp.float32)),
        mesh=_MESH,
        scratch_types=[pltpu.VMEM((1, CH), jnp.int32),
                       pltpu.VMEM((CH, C), jnp.float32)],
    )
    return f(Q, K, dstR, srcR)


def _sc_denom_body(ex16_hbm, dstR_hbm, z128_hbm, dpart_hbm,
                   ibuf, exbuf, padbuf, acc):
    c = lax.axis_index("c")
    s = lax.axis_index("s")
    row0 = s * ROWS_PER_TILE
    pltpu.sync_copy(z128_hbm, acc.at[pl.ds(row0, ROWS_PER_TILE)])
    pltpu.sync_copy(z128_hbm.at[pl.ds(0, CH)], padbuf)
    plsc.subcore_barrier()

    lo = c * 625 + s * 39 + jnp.minimum(s, 1)
    cnt = 39 + jnp.where(s < 1, 1, 0)

    @pl.loop(0, cnt)
    def _(i):
        r = lo + i
        pltpu.sync_copy(dstR_hbm.at[r], ibuf)
        pltpu.sync_copy(ex16_hbm.at[pl.ds(r * CH, CH)], exbuf)

        @pl.loop(0, CH)
        def _(j):
            padbuf[j, pl.ds(0, 16)] = exbuf[j, :]

        pltpu.sync_copy(padbuf, acc.at[ibuf.at[0]], add=True)

    plsc.subcore_barrier()
    pltpu.sync_copy(acc.at[pl.ds(row0, ROWS_PER_TILE)],
                    dpart_hbm.at[c, pl.ds(row0, ROWS_PER_TILE)])


def _sc_denom(ex16, dstR, z128):
    f = pl.kernel(
        _sc_denom_body,
        out_type=jax.ShapeDtypeStruct((NC, NP, 128), jnp.float32),
        mesh=_MESH,
        scratch_types=[pltpu.VMEM((1, CH), jnp.int32),
                       pltpu.VMEM((CH, 16), jnp.float32),
                       pltpu.VMEM((CH, 128), jnp.float32),
                       pltpu.VMEM_SHARED((NP, 128), jnp.float32)],
    )
    return f(ex16, dstR, z128)


_GDN = lax.GatherDimensionNumbers(
    offset_dims=(), collapsed_slice_dims=(0,), start_index_map=(0,))


def _splat(row, col):
    idx = jnp.full((16, 1), col, jnp.int32)
    return lax.gather(row, idx, _GDN, (1,),
                      mode=lax.GatherScatterMode.PROMISE_IN_BOUNDS)


def _sc_msg_body(VR_hbm, ex16_hbm, dstR_hbm, srcR_hbm, z128_hbm,
                 rst_hbm, dbuf, sbuf, i2buf, exbuf, vbuf, acc, lsem, vsem):
    c = lax.axis_index("c")
    s = lax.axis_index("s")
    row0 = s * ROWS_PER_TILE
    lo = s * 78 + jnp.minimum(s, 2)
    cnt = 78 + jnp.where(s < 2, 1, 0)

    def start_loads(k, r):
        pltpu.make_async_copy(dstR_hbm.at[r], dbuf.at[k], lsem.at[k, 0]).start()
        pltpu.make_async_copy(srcR_hbm.at[r], sbuf.at[k], lsem.at[k, 1]).start()
        pltpu.make_async_copy(ex16_hbm.at[pl.ds(r * CH, CH)], exbuf.at[k],
                              lsem.at[k, 2]).start()

    def wait_loads(k, r):
        pltpu.make_async_copy(dstR_hbm.at[r], dbuf.at[k], lsem.at[k, 0]).wait()
        pltpu.make_async_copy(srcR_hbm.at[r], sbuf.at[k], lsem.at[k, 1]).wait()
        pltpu.make_async_copy(ex16_hbm.at[pl.ds(r * CH, CH)], exbuf.at[k],
                              lsem.at[k, 2]).wait()

    for p in range(2):
        g = c * 2 + p
        pltpu.sync_copy(z128_hbm, acc.at[pl.ds(row0, ROWS_PER_TILE)])
        plsc.subcore_barrier()

        def start_v(k):
            for j in range(CH // 16):
                sv = sbuf[k, 0, pl.ds(j * 16, 16)]
                i2buf[0, pl.ds(j * 16, 16)] = sv * 4 + g
            pltpu.make_async_copy(VR_hbm.at[i2buf.at[0]], vbuf, vsem).start()

        def finish(k):
            pltpu.make_async_copy(VR_hbm.at[i2buf.at[0]], vbuf, vsem).wait()

            @pl.loop(0, CH)
            def _(j):
                row = exbuf[k, j, :]
                a0 = _splat(row, 2 * g)
                a1 = _splat(row, 2 * g + 1)
                for t in range(4):
                    sl = pl.ds(t * 16, 16)
                    vbuf[j, sl] = vbuf[j, sl] * a0
                for t in range(4, 8):
                    sl = pl.ds(t * 16, 16)
                    vbuf[j, sl] = vbuf[j, sl] * a1

            pltpu.sync_copy(vbuf, acc.at[dbuf.at[k, 0]], add=True)

        npairs = cnt // 2
        start_loads(0, lo)

        @pl.loop(0, npairs)
        def _(ip):
            ia = lo + 2 * ip
            wait_loads(0, ia)
            start_v(0)
            start_loads(1, ia + 1)
            finish(0)
            wait_loads(1, ia + 1)
            start_v(1)

            @pl.when(2 * ip + 2 < cnt)
            def _():
                start_loads(0, ia + 2)

            finish(1)

        @pl.when(cnt % 2 == 1)
        def _():
            rl = lo + cnt - 1
            wait_loads(0, rl)
            start_v(0)
            finish(0)

        plsc.subcore_barrier()
        pltpu.sync_copy(acc.at[pl.ds(row0, ROWS_PER_TILE)],
                        rst_hbm.at[pl.ds(row0, ROWS_PER_TILE),
                                   pl.ds(g * 128, 128)])
        plsc.subcore_barrier()


def _sc_msg(VR, ex16, dstR, srcR, z128):
    f = pl.kernel(
        _sc_msg_body,
        out_type=jax.ShapeDtypeStruct((NP, C), jnp.float32),
        mesh=_MESH,
        scratch_types=[pltpu.VMEM((2, 1, CH), jnp.int32),
                       pltpu.VMEM((2, 1, CH), jnp.int32),
                       pltpu.VMEM((1, CH), jnp.int32),
                       pltpu.VMEM((2, CH, 16), jnp.float32),
                       pltpu.VMEM((CH, 128), jnp.float32),
                       pltpu.VMEM_SHARED((NP, 128), jnp.float32),
                       pltpu.SemaphoreType.DMA((2, 3)),
                       pltpu.SemaphoreType.DMA],
    )
    return f(VR, ex16, dstR, srcR, z128)


def kernel(q, k, v, edge_feat, edge_index, Wq, Wk, Wv, We, be, Wo, Wres,
           W1, bf1, W2, bf2, gamma1, beta1, gamma2, beta2):
    src = edge_index[0].astype(jnp.int32)
    dst = edge_index[1].astype(jnp.int32)
    pad = jnp.zeros((E_PAD - E,), jnp.int32)
    dstR = jnp.concatenate([dst, pad]).reshape(NCH, 1, CH)
    srcR = jnp.concatenate([src, pad]).reshape(NCH, 1, CH)
    efP = jnp.concatenate(
        [edge_feat, jnp.zeros((E_PAD - E, 16), jnp.float32)], axis=0)

    WeP = We
    WeP = jnp.concatenate([We, jnp.zeros((16, 8), jnp.float32)], axis=1)
    beP = jnp.concatenate([be, jnp.full((8,), -1e30, jnp.float32)])
    beP = beP.reshape(1, 16)

    m_np = np.zeros((C, 16), np.float32)
    for h in range(HEADS):
        m_np[h * OUT_DIM:(h + 1) * OUT_DIM, h] = 1.0
    M = jnp.asarray(m_np)

    z128 = jnp.zeros((ROWS_PER_TILE, 128), jnp.float32)

    Q, K, V, Qres = _proj(q, k, v, Wq, Wk, Wv, Wres)
    VR = V.reshape(N * 4, 128)

    Qg, Kg = _sc_gather(Q, K, dstR, srcR)
    ex16 = _scores(Qg, Kg, efP, WeP, beP, M)

    dpart = _sc_denom(ex16, dstR, z128)
    rden = _rden(dpart)

    rst = _sc_msg(VR, ex16, dstR, srcR, z128)
    rst = rst[:N]
    rdenN = rden[:N]

    Z, s1, q1 = _head1(rst, rdenN, Wo, Qres)
    mean1 = s1 / N
    var1 = q1 / N - mean1 * mean1
    a1 = gamma1 / jnp.sqrt(var1 + 1e-5)
    b1 = beta1 - mean1 * a1

    Y, s2, q2 = _head2(Z, a1, b1, W1, bf1.reshape(1, -1), W2,
                       bf2.reshape(1, -1))
    mean2 = s2 / N
    var2 = q2 / N - mean2 * mean2
    a2 = gamma2 / jnp.sqrt(var2 + 1e-5)
    b2 = beta2 - mean2 * a2

    return _bn2(Y, a2, b2)


# speedup vs baseline: 1.9114x; 1.0415x over previous
import dataclasses
import functools

import jax
import jax.numpy as jnp
import numpy as np
from jax import lax
from jax.experimental import pallas as pl
from jax.experimental.pallas import tpu as pltpu
from jax.experimental.pallas import tpu_sc as plsc

N = 10000
E = 160000
IN_DIM = 256
OUT_DIM = 64
HEADS = 8
C = OUT_DIM * HEADS

CH = 128
E_PAD = 160000
NCH = E_PAD // CH
NC, NS = 2, 16
NP = 10112
ROWS_PER_TILE = NP // NS

_MESH = plsc.VectorSubcoreMesh(core_axis_name="c", subcore_axis_name="s")

_SC_CP = pltpu.CompilerParams()
if "needs_layout_passes" in pltpu.CompilerParams.__dataclass_fields__:
    _SC_CP = dataclasses.replace(_SC_CP, needs_layout_passes=False)

_NBLK = 400
_EBLK = 2000


def _proj_body(q_ref, k_ref, v_ref, wq_ref, wk_ref, wv_ref, wr_ref,
               Q_ref, K_ref, V_ref, R_ref):
    Q_ref[...] = jnp.dot(q_ref[...], wq_ref[...],
                         preferred_element_type=jnp.float32)
    K_ref[...] = jnp.dot(k_ref[...], wk_ref[...],
                         preferred_element_type=jnp.float32)
    V_ref[...] = jnp.dot(v_ref[...], wv_ref[...],
                         preferred_element_type=jnp.float32)
    R_ref[...] = jnp.dot(q_ref[...], wr_ref[...],
                         preferred_element_type=jnp.float32)


def _proj(q, k, v, Wq, Wk, Wv, Wres):
    nb = N // _NBLK
    blk = lambda i: (i, 0)
    w_spec = pl.BlockSpec((IN_DIM, C), lambda i: (0, 0))
    return pl.pallas_call(
        _proj_body,
        grid=(nb,),
        in_specs=[pl.BlockSpec((_NBLK, IN_DIM), blk)] * 3 + [w_spec] * 4,
        out_specs=[pl.BlockSpec((_NBLK, C), blk)] * 4,
        out_shape=[jax.ShapeDtypeStruct((N, C), jnp.float32)] * 4,
    )(q, k, v, Wq, Wk, Wv, Wres)


def _scores_body(qg_ref, kg_ref, ef_ref, wep_ref, bep_ref, m_ref, ex16_ref):
    prod = qg_ref[...] * kg_ref[...]
    s = jnp.dot(prod, m_ref[...], preferred_element_type=jnp.float32)
    s = s * (1.0 / float(OUT_DIM) ** 0.5)
    s = s + jnp.dot(ef_ref[...], wep_ref[...],
                    preferred_element_type=jnp.float32) + bep_ref[...]
    row = (pl.program_id(0) * _EBLK
           + lax.broadcasted_iota(jnp.int32, (_EBLK, 1), 0))
    s = jnp.where(row < E, s, -1e30)
    ex16_ref[...] = jnp.exp(s)


def _scores(Qg, Kg, ef, WeP, beP, M):
    nb = E_PAD // _EBLK
    blk = lambda i: (i, 0)
    return pl.pallas_call(
        _scores_body,
        grid=(nb,),
        in_specs=[pl.BlockSpec((_EBLK, C), blk),
                  pl.BlockSpec((_EBLK, C), blk),
                  pl.BlockSpec((_EBLK, 16), blk),
                  pl.BlockSpec((16, 16), lambda i: (0, 0)),
                  pl.BlockSpec((1, 16), lambda i: (0, 0)),
                  pl.BlockSpec((C, 16), lambda i: (0, 0))],
        out_specs=pl.BlockSpec((_EBLK, 16), blk),
        out_shape=jax.ShapeDtypeStruct((E_PAD, 16), jnp.float32),
    )(Qg, Kg, ef, WeP, beP, M)


def _rden_body(dp_ref, out_ref):
    den = dp_ref[0] + dp_ref[1]
    out_ref[...] = 1.0 / jnp.maximum(den, 1e-30)


def _rden(dpart):
    return pl.pallas_call(
        _rden_body,
        out_shape=jax.ShapeDtypeStruct((NP, 128), jnp.float32),
    )(dpart)


def _head1_body(rst_ref, rd_ref, wo_ref, qres_ref, z_ref, s_ref, q_ref):
    rd = rd_ref[...]
    rscale = jnp.concatenate(
        [jnp.broadcast_to(rd[:, h:h + 1], (_NBLK, OUT_DIM))
         for h in range(HEADS)], axis=1)
    z = jnp.dot(rst_ref[...] * rscale, wo_ref[...],
                preferred_element_type=jnp.float32) + qres_ref[...]
    z_ref[...] = z

    @pl.when(pl.program_id(0) == 0)
    def _():
        s_ref[...] = jnp.zeros_like(s_ref)
        q_ref[...] = jnp.zeros_like(q_ref)

    s_ref[...] += jnp.sum(z, axis=0, keepdims=True)
    q_ref[...] += jnp.sum(z * z, axis=0, keepdims=True)


def _head1(rst, rdenN, Wo, Qres):
    nb = N // _NBLK
    blk = lambda i: (i, 0)
    acc = pl.BlockSpec((1, C), lambda i: (0, 0))
    return pl.pallas_call(
        _head1_body,
        grid=(nb,),
        in_specs=[pl.BlockSpec((_NBLK, C), blk),
                  pl.BlockSpec((_NBLK, 128), blk),
                  pl.BlockSpec((C, C), lambda i: (0, 0)),
                  pl.BlockSpec((_NBLK, C), blk)],
        out_specs=[pl.BlockSpec((_NBLK, C), blk), acc, acc],
        out_shape=[jax.ShapeDtypeStruct((N, C), jnp.float32),
                   jax.ShapeDtypeStruct((1, C), jnp.float32),
                   jax.ShapeDtypeStruct((1, C), jnp.float32)],
    )(rst, rdenN, Wo, Qres)


def _head2_body(z_ref, a1_ref, b1_ref, w1_ref, bf1_ref, w2_ref, bf2_ref,
                y_ref, s_ref, q_ref):
    zn = z_ref[...] * a1_ref[...] + b1_ref[...]
    h = jnp.maximum(jnp.dot(zn, w1_ref[...],
                            preferred_element_type=jnp.float32)
                    + bf1_ref[...], 0.0)
    y = jnp.dot(h, w2_ref[...],
                preferred_element_type=jnp.float32) + bf2_ref[...] + zn
    y_ref[...] = y

    @pl.when(pl.program_id(0) == 0)
    def _():
        s_ref[...] = jnp.zeros_like(s_ref)
        q_ref[...] = jnp.zeros_like(q_ref)

    s_ref[...] += jnp.sum(y, axis=0, keepdims=True)
    q_ref[...] += jnp.sum(y * y, axis=0, keepdims=True)


def _head2(Z, a1, b1, W1, bf1, W2, bf2):
    nb = N // _NBLK
    blk = lambda i: (i, 0)
    acc = pl.BlockSpec((1, C), lambda i: (0, 0))
    one = lambda shape: pl.BlockSpec(shape, lambda i: (0, 0))
    return pl.pallas_call(
        _head2_body,
        grid=(nb,),
        in_specs=[pl.BlockSpec((_NBLK, C), blk),
                  one((1, C)), one((1, C)),
                  one((C, 2 * C)), one((1, 2 * C)),
                  one((2 * C, C)), one((1, C))],
        out_specs=[pl.BlockSpec((_NBLK, C), blk), acc, acc],
        out_shape=[jax.ShapeDtypeStruct((N, C), jnp.float32),
                   jax.ShapeDtypeStruct((1, C), jnp.float32),
                   jax.ShapeDtypeStruct((1, C), jnp.float32)],
    )(Z, a1, b1, W1, bf1, W2, bf2)


def _bn2_body(y_ref, a2_ref, b2_ref, out_ref):
    out_ref[...] = y_ref[...] * a2_ref[...] + b2_ref[...]


def _bn2(Y, a2, b2):
    nb = N // _NBLK
    blk = lambda i: (i, 0)
    return pl.pallas_call(
        _bn2_body,
        grid=(nb,),
        in_specs=[pl.BlockSpec((_NBLK, C), blk),
                  pl.BlockSpec((1, C), lambda i: (0, 0)),
                  pl.BlockSpec((1, C), lambda i: (0, 0))],
        out_specs=pl.BlockSpec((_NBLK, C), blk),
        out_shape=jax.ShapeDtypeStruct((N, C), jnp.float32),
    )(Y, a2, b2)


def _sc_gather_body(Q_hbm, K_hbm, dstR2_hbm, srcR2_hbm, Qg_hbm, Kg_hbm,
                    ibuf, bufA, bufB, gsem, wsem):
    c = lax.axis_index("c")
    s = lax.axis_index("s")
    w = s * NC + c
    ww = jnp.where(w < 16, w, w - 16)
    lo = ww * 78 + jnp.minimum(ww, 2)
    cnt = 78 + jnp.where(ww < 2, 1, 0)

    def run(tab_hbm, idxR_hbm, out_hbm):
        def gdesc(h):
            buf = bufA if h == 0 else bufB
            return pltpu.make_async_copy(tab_hbm.at[ibuf.at[h]], buf,
                                         gsem.at[h])

        def wdesc(h, r):
            buf = bufA if h == 0 else bufB
            return pltpu.make_async_copy(
                buf, out_hbm.at[pl.ds(r * CH + h * 64, 64)], wsem.at[h])

        @pl.loop(0, cnt)
        def _(i):
            r = lo + i
            pltpu.sync_copy(idxR_hbm.at[r], ibuf)

            @pl.when(i > 0)
            def _():
                wdesc(0, r - 1).wait()
                wdesc(1, r - 1).wait()

            gdesc(0).start()
            gdesc(1).start()
            gdesc(0).wait()
            wdesc(0, r).start()
            gdesc(1).wait()
            wdesc(1, r).start()

        last = lo + cnt - 1
        wdesc(0, last).wait()
        wdesc(1, last).wait()

    @pl.when(w < 16)
    def _():
        run(Q_hbm, dstR2_hbm, Qg_hbm)

    @pl.when(w >= 16)
    def _():
        run(K_hbm, srcR2_hbm, Kg_hbm)


def _sc_gather(Q, K, dstR2, srcR2):
    f = pl.kernel(
        _sc_gather_body,
        out_type=(jax.ShapeDtypeStruct((E_PAD, C), jnp.float32),
                  jax.ShapeDtypeStruct((E_PAD, C), jnp.float32)),
        mesh=_MESH,
        scratch_types=[pltpu.VMEM((2, 64), jnp.int32),
                       pltpu.VMEM((64, C), jnp.float32),
                       pltpu.VMEM((64, C), jnp.float32),
                       pltpu.SemaphoreType.DMA((2,)),
                       pltpu.SemaphoreType.DMA((2,))],
    )
    return f(Q, K, dstR2, srcR2)


def _sc_denom_body(ex16_hbm, dstR_hbm, z128_hbm, dpart_hbm,
                   ibuf, exbuf, padbuf, acc):
    c = lax.axis_index("c")
    s = lax.axis_index("s")
    row0 = s * ROWS_PER_TILE
    pltpu.sync_copy(z128_hbm, acc.at[pl.ds(row0, ROWS_PER_TILE)])
    pltpu.sync_copy(z128_hbm.at[pl.ds(0, CH)], padbuf)
    plsc.subcore_barrier()

    lo = c * 625 + s * 39 + jnp.minimum(s, 1)
    cnt = 39 + jnp.where(s < 1, 1, 0)

    @pl.loop(0, cnt)
    def _(i):
        r = lo + i
        pltpu.sync_copy(dstR_hbm.at[r], ibuf)
        pltpu.sync_copy(ex16_hbm.at[pl.ds(r * CH, CH)], exbuf)

        @pl.loop(0, CH)
        def _(j):
            padbuf[j, pl.ds(0, 16)] = exbuf[j, :]

        pltpu.sync_copy(padbuf, acc.at[ibuf.at[0]], add=True)

    plsc.subcore_barrier()
    pltpu.sync_copy(acc.at[pl.ds(row0, ROWS_PER_TILE)],
                    dpart_hbm.at[c, pl.ds(row0, ROWS_PER_TILE)])


def _sc_denom(ex16, dstR, z128):
    f = pl.kernel(
        _sc_denom_body,
        out_type=jax.ShapeDtypeStruct((NC, NP, 128), jnp.float32),
        mesh=_MESH,
        scratch_types=[pltpu.VMEM((1, CH), jnp.int32),
                       pltpu.VMEM((CH, 16), jnp.float32),
                       pltpu.VMEM((CH, 128), jnp.float32),
                       pltpu.VMEM_SHARED((NP, 128), jnp.float32)],
    )
    return f(ex16, dstR, z128)


_GDN = lax.GatherDimensionNumbers(
    offset_dims=(), collapsed_slice_dims=(0,), start_index_map=(0,))


def _splat(row, col):
    idx = jnp.full((16, 1), col, jnp.int32)
    return lax.gather(row, idx, _GDN, (1,),
                      mode=lax.GatherScatterMode.PROMISE_IN_BOUNDS)


def _sc_msg_body(VR_hbm, ex16_hbm, dstR_hbm, srcR_hbm, z128_hbm,
                 rst_hbm, dbuf, sbuf, i2buf, exbuf, vbuf, acc, lsem, vsem):
    c = lax.axis_index("c")
    s = lax.axis_index("s")
    row0 = s * ROWS_PER_TILE
    lo = s * 78 + jnp.minimum(s, 2)
    cnt = 78 + jnp.where(s < 2, 1, 0)

    def start_loads(k, r):
        pltpu.make_async_copy(dstR_hbm.at[r], dbuf.at[k], lsem.at[k, 0]).start()
        pltpu.make_async_copy(srcR_hbm.at[r], sbuf.at[k], lsem.at[k, 1]).start()
        pltpu.make_async_copy(ex16_hbm.at[pl.ds(r * CH, CH)], exbuf.at[k],
                              lsem.at[k, 2]).start()

    def wait_loads(k, r):
        pltpu.make_async_copy(dstR_hbm.at[r], dbuf.at[k], lsem.at[k, 0]).wait()
        pltpu.make_async_copy(srcR_hbm.at[r], sbuf.at[k], lsem.at[k, 1]).wait()
        pltpu.make_async_copy(ex16_hbm.at[pl.ds(r * CH, CH)], exbuf.at[k],
                              lsem.at[k, 2]).wait()

    for p in range(2):
        g = c * 2 + p
        pltpu.sync_copy(z128_hbm, acc.at[pl.ds(row0, ROWS_PER_TILE)])
        plsc.subcore_barrier()

        def start_v(k):
            for j in range(CH // 16):
                sv = sbuf[k, 0, pl.ds(j * 16, 16)]
                i2buf[0, pl.ds(j * 16, 16)] = sv * 4 + g
            pltpu.make_async_copy(VR_hbm.at[i2buf.at[0]], vbuf, vsem).start()

        def finish(k):
            pltpu.make_async_copy(VR_hbm.at[i2buf.at[0]], vbuf, vsem).wait()

            @pl.loop(0, CH)
            def _(j):
                row = exbuf[k, j, :]
                a0 = _splat(row, 2 * g)
                a1 = _splat(row, 2 * g + 1)
                for t in range(4):
                    sl = pl.ds(t * 16, 16)
                    vbuf[j, sl] = vbuf[j, sl] * a0
                for t in range(4, 8):
                    sl = pl.ds(t * 16, 16)
                    vbuf[j, sl] = vbuf[j, sl] * a1

            pltpu.sync_copy(vbuf, acc.at[dbuf.at[k, 0]], add=True)

        npairs = cnt // 2
        start_loads(0, lo)

        @pl.loop(0, npairs)
        def _(ip):
            ia = lo + 2 * ip
            wait_loads(0, ia)
            start_v(0)
            start_loads(1, ia + 1)
            finish(0)
            wait_loads(1, ia + 1)
            start_v(1)

            @pl.when(2 * ip + 2 < cnt)
            def _():
                start_loads(0, ia + 2)

            finish(1)

        @pl.when(cnt % 2 == 1)
        def _():
            rl = lo + cnt - 1
            wait_loads(0, rl)
            start_v(0)
            finish(0)

        plsc.subcore_barrier()
        pltpu.sync_copy(acc.at[pl.ds(row0, ROWS_PER_TILE)],
                        rst_hbm.at[pl.ds(row0, ROWS_PER_TILE),
                                   pl.ds(g * 128, 128)])
        plsc.subcore_barrier()


def _sc_msg(VR, ex16, dstR, srcR, z128):
    f = pl.kernel(
        _sc_msg_body,
        out_type=jax.ShapeDtypeStruct((NP, C), jnp.float32),
        mesh=_MESH,
        scratch_types=[pltpu.VMEM((2, 1, CH), jnp.int32),
                       pltpu.VMEM((2, 1, CH), jnp.int32),
                       pltpu.VMEM((1, CH), jnp.int32),
                       pltpu.VMEM((2, CH, 16), jnp.float32),
                       pltpu.VMEM((CH, 128), jnp.float32),
                       pltpu.VMEM_SHARED((NP, 128), jnp.float32),
                       pltpu.SemaphoreType.DMA((2, 3)),
                       pltpu.SemaphoreType.DMA],
    )
    return f(VR, ex16, dstR, srcR, z128)


def kernel(q, k, v, edge_feat, edge_index, Wq, Wk, Wv, We, be, Wo, Wres,
           W1, bf1, W2, bf2, gamma1, beta1, gamma2, beta2):
    src = edge_index[0].astype(jnp.int32)
    dst = edge_index[1].astype(jnp.int32)
    pad = jnp.zeros((E_PAD - E,), jnp.int32)
    dstR = jnp.concatenate([dst, pad]).reshape(NCH, 1, CH)
    srcR = jnp.concatenate([src, pad]).reshape(NCH, 1, CH)
    dstR2 = dstR.reshape(NCH, 2, 64)
    srcR2 = srcR.reshape(NCH, 2, 64)
    efP = jnp.concatenate(
        [edge_feat, jnp.zeros((E_PAD - E, 16), jnp.float32)], axis=0)

    WeP = We
    WeP = jnp.concatenate([We, jnp.zeros((16, 8), jnp.float32)], axis=1)
    beP = jnp.concatenate([be, jnp.full((8,), -1e30, jnp.float32)])
    beP = beP.reshape(1, 16)

    m_np = np.zeros((C, 16), np.float32)
    for h in range(HEADS):
        m_np[h * OUT_DIM:(h + 1) * OUT_DIM, h] = 1.0
    M = jnp.asarray(m_np)

    z128 = jnp.zeros((ROWS_PER_TILE, 128), jnp.float32)

    Q, K, V, Qres = _proj(q, k, v, Wq, Wk, Wv, Wres)
    VR = V.reshape(N * 4, 128)

    Qg, Kg = _sc_gather(Q, K, dstR2, srcR2)
    ex16 = _scores(Qg, Kg, efP, WeP, beP, M)

    dpart = _sc_denom(ex16, dstR, z128)
    rden = _rden(dpart)

    rst = _sc_msg(VR, ex16, dstR, srcR, z128)
    rst = rst[:N]
    rdenN = rden[:N]

    Z, s1, q1 = _head1(rst, rdenN, Wo, Qres)
    mean1 = s1 / N
    var1 = q1 / N - mean1 * mean1
    a1 = gamma1 / jnp.sqrt(var1 + 1e-5)
    b1 = beta1 - mean1 * a1

    Y, s2, q2 = _head2(Z, a1, b1, W1, bf1.reshape(1, -1), W2,
                       bf2.reshape(1, -1))
    mean2 = s2 / N
    var2 = q2 / N - mean2 * mean2
    a2 = gamma2 / jnp.sqrt(var2 + 1e-5)
    b2 = beta2 - mean2 * a2

    return _bn2(Y, a2, b2)


# speedup vs baseline: 1.9133x; 1.0010x over previous
import jax
import jax.numpy as jnp
import numpy as np
from jax import lax
from jax.experimental import pallas as pl
from jax.experimental.pallas import tpu as pltpu
from jax.experimental.pallas import tpu_sc as plsc

N = 10000
E = 160000
IN_DIM = 256
OUT_DIM = 64
HEADS = 8
C = OUT_DIM * HEADS

CH = 128
E_PAD = 160000
NCH = E_PAD // CH
NC, NS = 2, 16
NP = 10112
ROWS_PER_TILE = NP // NS

_MESH = plsc.VectorSubcoreMesh(core_axis_name="c", subcore_axis_name="s")

_NBLK = 400
_EBLK = 2000


def _proj_body(q_ref, k_ref, v_ref, wq_ref, wk_ref, wv_ref, wr_ref,
               Q_ref, K_ref, V_ref, R_ref):
    Q_ref[...] = jnp.dot(q_ref[...], wq_ref[...],
                         preferred_element_type=jnp.float32)
    K_ref[...] = jnp.dot(k_ref[...], wk_ref[...],
                         preferred_element_type=jnp.float32)
    V_ref[...] = jnp.dot(v_ref[...], wv_ref[...],
                         preferred_element_type=jnp.float32)
    R_ref[...] = jnp.dot(q_ref[...], wr_ref[...],
                         preferred_element_type=jnp.float32)


def _proj(q, k, v, Wq, Wk, Wv, Wres):
    nb = N // _NBLK
    blk = lambda i: (i, 0)
    w_spec = pl.BlockSpec((IN_DIM, C), lambda i: (0, 0))
    return pl.pallas_call(
        _proj_body,
        grid=(nb,),
        in_specs=[pl.BlockSpec((_NBLK, IN_DIM), blk)] * 3 + [w_spec] * 4,
        out_specs=[pl.BlockSpec((_NBLK, C), blk)] * 4,
        out_shape=[jax.ShapeDtypeStruct((N, C), jnp.float32)] * 4,
    )(q, k, v, Wq, Wk, Wv, Wres)


def _scores_body(qg_ref, kg_ref, ef_ref, wep_ref, bep_ref, m_ref, ex16_ref):
    prod = qg_ref[...] * kg_ref[...]
    s = jnp.dot(prod, m_ref[...], preferred_element_type=jnp.float32)
    s = s * (1.0 / float(OUT_DIM) ** 0.5)
    s = s + jnp.dot(ef_ref[...], wep_ref[...],
                    preferred_element_type=jnp.float32) + bep_ref[...]
    row = (pl.program_id(0) * _EBLK
           + lax.broadcasted_iota(jnp.int32, (_EBLK, 1), 0))
    s = jnp.where(row < E, s, -1e30)
    ex16_ref[...] = jnp.exp(s)


def _scores(Qg, Kg, ef, WeP, beP, M):
    nb = E_PAD // _EBLK
    blk = lambda i: (i, 0)
    return pl.pallas_call(
        _scores_body,
        grid=(nb,),
        in_specs=[pl.BlockSpec((_EBLK, C), blk),
                  pl.BlockSpec((_EBLK, C), blk),
                  pl.BlockSpec((_EBLK, 16), blk),
                  pl.BlockSpec((16, 16), lambda i: (0, 0)),
                  pl.BlockSpec((1, 16), lambda i: (0, 0)),
                  pl.BlockSpec((C, 16), lambda i: (0, 0))],
        out_specs=pl.BlockSpec((_EBLK, 16), blk),
        out_shape=jax.ShapeDtypeStruct((E_PAD, 16), jnp.float32),
    )(Qg, Kg, ef, WeP, beP, M)


def _rden_body(dp_ref, out_ref):
    den = dp_ref[0] + dp_ref[1]
    out_ref[...] = 1.0 / jnp.maximum(den, 1e-30)


def _rden(dpart):
    return pl.pallas_call(
        _rden_body,
        out_shape=jax.ShapeDtypeStruct((NP, 128), jnp.float32),
    )(dpart)


def _head1_body(rst_ref, rd_ref, wo_ref, qres_ref, z_ref, s_ref, q_ref):
    rd = rd_ref[...]
    rscale = jnp.concatenate(
        [jnp.broadcast_to(rd[:, h:h + 1], (_NBLK, OUT_DIM))
         for h in range(HEADS)], axis=1)
    z = jnp.dot(rst_ref[...] * rscale, wo_ref[...],
                preferred_element_type=jnp.float32) + qres_ref[...]
    z_ref[...] = z

    @pl.when(pl.program_id(0) == 0)
    def _():
        s_ref[...] = jnp.zeros_like(s_ref)
        q_ref[...] = jnp.zeros_like(q_ref)

    s_ref[...] += jnp.sum(z, axis=0, keepdims=True)
    q_ref[...] += jnp.sum(z * z, axis=0, keepdims=True)


def _head1(rst, rdenN, Wo, Qres):
    nb = N // _NBLK
    blk = lambda i: (i, 0)
    acc = pl.BlockSpec((1, C), lambda i: (0, 0))
    return pl.pallas_call(
        _head1_body,
        grid=(nb,),
        in_specs=[pl.BlockSpec((_NBLK, C), blk),
                  pl.BlockSpec((_NBLK, 128), blk),
                  pl.BlockSpec((C, C), lambda i: (0, 0)),
                  pl.BlockSpec((_NBLK, C), blk)],
        out_specs=[pl.BlockSpec((_NBLK, C), blk), acc, acc],
        out_shape=[jax.ShapeDtypeStruct((N, C), jnp.float32),
                   jax.ShapeDtypeStruct((1, C), jnp.float32),
                   jax.ShapeDtypeStruct((1, C), jnp.float32)],
    )(rst, rdenN, Wo, Qres)


def _head2_body(z_ref, a1_ref, b1_ref, w1_ref, bf1_ref, w2_ref, bf2_ref,
                y_ref, s_ref, q_ref):
    zn = z_ref[...] * a1_ref[...] + b1_ref[...]
    h = jnp.maximum(jnp.dot(zn, w1_ref[...],
                            preferred_element_type=jnp.float32)
                    + bf1_ref[...], 0.0)
    y = jnp.dot(h, w2_ref[...],
                preferred_element_type=jnp.float32) + bf2_ref[...] + zn
    y_ref[...] = y

    @pl.when(pl.program_id(0) == 0)
    def _():
        s_ref[...] = jnp.zeros_like(s_ref)
        q_ref[...] = jnp.zeros_like(q_ref)

    s_ref[...] += jnp.sum(y, axis=0, keepdims=True)
    q_ref[...] += jnp.sum(y * y, axis=0, keepdims=True)


def _head2(Z, a1, b1, W1, bf1, W2, bf2):
    nb = N // _NBLK
    blk = lambda i: (i, 0)
    acc = pl.BlockSpec((1, C), lambda i: (0, 0))
    one = lambda shape: pl.BlockSpec(shape, lambda i: (0, 0))
    return pl.pallas_call(
        _head2_body,
        grid=(nb,),
        in_specs=[pl.BlockSpec((_NBLK, C), blk),
                  one((1, C)), one((1, C)),
                  one((C, 2 * C)), one((1, 2 * C)),
                  one((2 * C, C)), one((1, C))],
        out_specs=[pl.BlockSpec((_NBLK, C), blk), acc, acc],
        out_shape=[jax.ShapeDtypeStruct((N, C), jnp.float32),
                   jax.ShapeDtypeStruct((1, C), jnp.float32),
                   jax.ShapeDtypeStruct((1, C), jnp.float32)],
    )(Z, a1, b1, W1, bf1, W2, bf2)


def _bn2_body(y_ref, a2_ref, b2_ref, out_ref):
    out_ref[...] = y_ref[...] * a2_ref[...] + b2_ref[...]


def _bn2(Y, a2, b2):
    nb = N // _NBLK
    blk = lambda i: (i, 0)
    return pl.pallas_call(
        _bn2_body,
        grid=(nb,),
        in_specs=[pl.BlockSpec((_NBLK, C), blk),
                  pl.BlockSpec((1, C), lambda i: (0, 0)),
                  pl.BlockSpec((1, C), lambda i: (0, 0))],
        out_specs=pl.BlockSpec((_NBLK, C), blk),
        out_shape=jax.ShapeDtypeStruct((N, C), jnp.float32),
    )(Y, a2, b2)


def _sc_gather_body(Q_hbm, K_hbm, dstR2_hbm, srcR2_hbm, Qg_hbm, Kg_hbm,
                    ibuf, bufA, bufB, gsem, wsem):
    c = lax.axis_index("c")
    s = lax.axis_index("s")
    w = s * NC + c
    ww = jnp.where(w < 16, w, w - 16)
    lo = ww * 78 + jnp.minimum(ww, 2)
    cnt = 78 + jnp.where(ww < 2, 1, 0)

    def run(tab_hbm, idxR_hbm, out_hbm):
        def gdesc(h):
            buf = bufA if h == 0 else bufB
            return pltpu.make_async_copy(tab_hbm.at[ibuf.at[h]], buf,
                                         gsem.at[h])

        def wdesc(h, r):
            buf = bufA if h == 0 else bufB
            return pltpu.make_async_copy(
                buf, out_hbm.at[pl.ds(r * CH + h * 64, 64)], wsem.at[h])

        @pl.loop(0, cnt)
        def _(i):
            r = lo + i
            pltpu.sync_copy(idxR_hbm.at[r], ibuf)

            @pl.when(i > 0)
            def _():
                wdesc(0, r - 1).wait()
                wdesc(1, r - 1).wait()

            gdesc(0).start()
            gdesc(1).start()
            gdesc(0).wait()
            wdesc(0, r).start()
            gdesc(1).wait()
            wdesc(1, r).start()

        last = lo + cnt - 1
        wdesc(0, last).wait()
        wdesc(1, last).wait()

    @pl.when(w < 16)
    def _():
        run(Q_hbm, dstR2_hbm, Qg_hbm)

    @pl.when(w >= 16)
    def _():
        run(K_hbm, srcR2_hbm, Kg_hbm)


def _sc_gather(Q, K, dstR2, srcR2):
    f = pl.kernel(
        _sc_gather_body,
        out_type=(jax.ShapeDtypeStruct((E_PAD, C), jnp.float32),
                  jax.ShapeDtypeStruct((E_PAD, C), jnp.float32)),
        mesh=_MESH,
        scratch_types=[pltpu.VMEM((2, 64), jnp.int32),
                       pltpu.VMEM((64, C), jnp.float32),
                       pltpu.VMEM((64, C), jnp.float32),
                       pltpu.SemaphoreType.DMA((2,)),
                       pltpu.SemaphoreType.DMA((2,))],
    )
    return f(Q, K, dstR2, srcR2)


def _sc_denom_body(ex16_hbm, dstR_hbm, z128_hbm, dpart_hbm,
                   ibuf, exbuf, padbuf, acc):
    c = lax.axis_index("c")
    s = lax.axis_index("s")
    row0 = s * ROWS_PER_TILE
    pltpu.sync_copy(z128_hbm, acc.at[pl.ds(row0, ROWS_PER_TILE)])
    pltpu.sync_copy(z128_hbm.at[pl.ds(0, CH)], padbuf)
    plsc.subcore_barrier()

    lo = c * 625 + s * 39 + jnp.minimum(s, 1)
    cnt = 39 + jnp.where(s < 1, 1, 0)

    @pl.loop(0, cnt)
    def _(i):
        r = lo + i
        pltpu.sync_copy(dstR_hbm.at[r], ibuf)
        pltpu.sync_copy(ex16_hbm.at[pl.ds(r * CH, CH)], exbuf)

        @pl.loop(0, CH)
        def _(j):
            padbuf[j, pl.ds(0, 16)] = exbuf[j, :]

        pltpu.sync_copy(padbuf, acc.at[ibuf.at[0]], add=True)

    plsc.subcore_barrier()
    pltpu.sync_copy(acc.at[pl.ds(row0, ROWS_PER_TILE)],
                    dpart_hbm.at[c, pl.ds(row0, ROWS_PER_TILE)])


def _sc_denom(ex16, dstR, z128):
    f = pl.kernel(
        _sc_denom_body,
        out_type=jax.ShapeDtypeStruct((NC, NP, 128), jnp.float32),
        mesh=_MESH,
        scratch_types=[pltpu.VMEM((1, CH), jnp.int32),
                       pltpu.VMEM((CH, 16), jnp.float32),
                       pltpu.VMEM((CH, 128), jnp.float32),
                       pltpu.VMEM_SHARED((NP, 128), jnp.float32)],
    )
    return f(ex16, dstR, z128)


_GDN = lax.GatherDimensionNumbers(
    offset_dims=(), collapsed_slice_dims=(0,), start_index_map=(0,))


def _splat(row, col):
    idx = jnp.full((16, 1), col, jnp.int32)
    return lax.gather(row, idx, _GDN, (1,),
                      mode=lax.GatherScatterMode.PROMISE_IN_BOUNDS)


def _sc_msg_body(VR_hbm, ex16_hbm, dstR_hbm, srcR_hbm, z128_hbm,
                 rst_hbm, dbuf, sbuf, i2buf, exbuf, vbuf, acc, lsem, vsem):
    c = lax.axis_index("c")
    s = lax.axis_index("s")
    row0 = s * ROWS_PER_TILE
    lo = s * 78 + jnp.minimum(s, 2)
    cnt = 78 + jnp.where(s < 2, 1, 0)

    def start_loads(k, r):
        pltpu.make_async_copy(dstR_hbm.at[r], dbuf.at[k], lsem.at[k, 0]).start()
        pltpu.make_async_copy(srcR_hbm.at[r], sbuf.at[k], lsem.at[k, 1]).start()
        pltpu.make_async_copy(ex16_hbm.at[pl.ds(r * CH, CH)], exbuf.at[k],
                              lsem.at[k, 2]).start()

    def wait_loads(k, r):
        pltpu.make_async_copy(dstR_hbm.at[r], dbuf.at[k], lsem.at[k, 0]).wait()
        pltpu.make_async_copy(srcR_hbm.at[r], sbuf.at[k], lsem.at[k, 1]).wait()
        pltpu.make_async_copy(ex16_hbm.at[pl.ds(r * CH, CH)], exbuf.at[k],
                              lsem.at[k, 2]).wait()

    for p in range(2):
        g = c * 2 + p
        pltpu.sync_copy(z128_hbm, acc.at[pl.ds(row0, ROWS_PER_TILE)])
        plsc.subcore_barrier()

        def start_v(k):
            for j in range(CH // 16):
                sv = sbuf[k, 0, pl.ds(j * 16, 16)]
                i2buf[0, pl.ds(j * 16, 16)] = sv * 4 + g
            pltpu.make_async_copy(VR_hbm.at[i2buf.at[0]], vbuf, vsem).start()

        def finish(k):
            pltpu.make_async_copy(VR_hbm.at[i2buf.at[0]], vbuf, vsem).wait()

            @pl.loop(0, CH)
            def _(j):
                row = exbuf[k, j, :]
                a0 = _splat(row, 2 * g)
                a1 = _splat(row, 2 * g + 1)
                for t in range(4):
                    sl = pl.ds(t * 16, 16)
                    vbuf[j, sl] = vbuf[j, sl] * a0
                for t in range(4, 8):
                    sl = pl.ds(t * 16, 16)
                    vbuf[j, sl] = vbuf[j, sl] * a1

            pltpu.sync_copy(vbuf, acc.at[dbuf.at[k, 0]], add=True)

        npairs = cnt // 2
        start_loads(0, lo)

        @pl.loop(0, npairs)
        def _(ip):
            ia = lo + 2 * ip
            wait_loads(0, ia)
            start_v(0)
            start_loads(1, ia + 1)
            finish(0)
            wait_loads(1, ia + 1)
            start_v(1)

            @pl.when(2 * ip + 2 < cnt)
            def _():
                start_loads(0, ia + 2)

            finish(1)

        @pl.when(cnt % 2 == 1)
        def _():
            rl = lo + cnt - 1
            wait_loads(0, rl)
            start_v(0)
            finish(0)

        plsc.subcore_barrier()
        pltpu.sync_copy(acc.at[pl.ds(row0, ROWS_PER_TILE)],
                        rst_hbm.at[pl.ds(row0, ROWS_PER_TILE),
                                   pl.ds(g * 128, 128)])
        plsc.subcore_barrier()


def _sc_msg(VR, ex16, dstR, srcR, z128):
    f = pl.kernel(
        _sc_msg_body,
        out_type=jax.ShapeDtypeStruct((NP, C), jnp.float32),
        mesh=_MESH,
        scratch_types=[pltpu.VMEM((2, 1, CH), jnp.int32),
                       pltpu.VMEM((2, 1, CH), jnp.int32),
                       pltpu.VMEM((1, CH), jnp.int32),
                       pltpu.VMEM((2, CH, 16), jnp.float32),
                       pltpu.VMEM((CH, 128), jnp.float32),
                       pltpu.VMEM_SHARED((NP, 128), jnp.float32),
                       pltpu.SemaphoreType.DMA((2, 3)),
                       pltpu.SemaphoreType.DMA],
    )
    return f(VR, ex16, dstR, srcR, z128)


def kernel(q, k, v, edge_feat, edge_index, Wq, Wk, Wv, We, be, Wo, Wres,
           W1, bf1, W2, bf2, gamma1, beta1, gamma2, beta2):
    src = edge_index[0].astype(jnp.int32)
    dst = edge_index[1].astype(jnp.int32)
    pad = jnp.zeros((E_PAD - E,), jnp.int32)
    dstR = jnp.concatenate([dst, pad]).reshape(NCH, 1, CH)
    srcR = jnp.concatenate([src, pad]).reshape(NCH, 1, CH)
    dstR2 = dstR.reshape(NCH, 2, 64)
    srcR2 = srcR.reshape(NCH, 2, 64)
    efP = jnp.concatenate(
        [edge_feat, jnp.zeros((E_PAD - E, 16), jnp.float32)], axis=0)

    WeP = jnp.concatenate([We, jnp.zeros((16, 8), jnp.float32)], axis=1)
    beP = jnp.concatenate([be, jnp.full((8,), -1e30, jnp.float32)])
    beP = beP.reshape(1, 16)

    m_np = np.zeros((C, 16), np.float32)
    for h in range(HEADS):
        m_np[h * OUT_DIM:(h + 1) * OUT_DIM, h] = 1.0
    M = jnp.asarray(m_np)

    z128 = jnp.zeros((ROWS_PER_TILE, 128), jnp.float32)

    Q, K, V, Qres = _proj(q, k, v, Wq, Wk, Wv, Wres)
    VR = V.reshape(N * 4, 128)

    Qg, Kg = _sc_gather(Q, K, dstR2, srcR2)
    ex16 = _scores(Qg, Kg, efP, WeP, beP, M)

    dpart = _sc_denom(ex16, dstR, z128)
    rden = _rden(dpart)

    rst = _sc_msg(VR, ex16, dstR, srcR, z128)
    rst = rst[:N]
    rdenN = rden[:N]

    Z, s1, q1 = _head1(rst, rdenN, Wo, Qres)
    mean1 = s1 / N
    var1 = q1 / N - mean1 * mean1
    a1 = gamma1 / jnp.sqrt(var1 + 1e-5)
    b1 = beta1 - mean1 * a1

    Y, s2, q2 = _head2(Z, a1, b1, W1, bf1.reshape(1, -1), W2,
                       bf2.reshape(1, -1))
    mean2 = s2 / N
    var2 = q2 / N - mean2 * mean2
    a2 = gamma2 / jnp.sqrt(var2 + 1e-5)
    b2 = beta2 - mean2 * a2

    return _bn2(Y, a2, b2)
